# Initial kernel scaffold; baseline (speedup 1.0000x reference)
import jax
import jax.numpy as jnp
from jax.experimental import pallas as pl


def kernel(x, edge_index, x_indices, c_indices, W1, b1, W2, b2, Wp, bp, emb_table):
    raise NotImplementedError("write your pallas kernel here")


# speedup vs baseline: 3.9261x; 3.9261x over previous
import functools

import jax
import jax.numpy as jnp
from jax import lax
from jax.experimental import pallas as pl
from jax.experimental.pallas import tpu as pltpu
from jax.experimental.pallas import tpu_sc as plsc

N_NODES = 10000
N_EDGES = 320000
D = 128
HID = 256
N_CELL = 100000
B = 4096

NC = 2
NS = 16
NW = NC * NS

NPAD = 10240
EPAD = 327680
TRASH = 10100
EW = EPAD // NW
ECH = EW // 128
GCH = 16
ROWS_PER_TILE = NPAD // NS

_mesh = plsc.VectorSubcoreMesh(core_axis_name="c", subcore_axis_name="s",
                               num_cores=NC, num_subcores=NS)
_f32 = jnp.float32
_sc_params = pltpu.CompilerParams(needs_layout_passes=False)


@functools.partial(
    pl.kernel,
    out_type=jax.ShapeDtypeStruct((NW, 2, NPAD), _f32),
    mesh=_mesh,
    compiler_params=_sc_params,
    scratch_types=[
        pltpu.VMEM((EW,), jnp.int32),
        pltpu.VMEM((EW,), jnp.int32),
        pltpu.VMEM((NPAD,), _f32),
        pltpu.VMEM((NPAD,), _f32),
    ],
)
def _deg_kernel(src_hbm, dst_hbm, deg_hbm, src_v, dst_v, dout_v, din_v):
    cid = lax.axis_index("c")
    sid = lax.axis_index("s")
    w = cid * NS + sid
    pltpu.sync_copy(src_hbm.at[pl.ds(w * EW, EW)], src_v)
    pltpu.sync_copy(dst_hbm.at[pl.ds(w * EW, EW)], dst_v)

    zeros = jnp.zeros((16,), _f32)

    def zbody(i, carry):
        dout_v[pl.ds(i * 16, 16)] = zeros
        din_v[pl.ds(i * 16, 16)] = zeros
        return carry

    lax.fori_loop(0, NPAD // 16, zbody, 0)

    ones = jnp.ones((16,), _f32)

    def body(i, carry):
        s = src_v[pl.ds(i * 16, 16)]
        d = dst_v[pl.ds(i * 16, 16)]
        plsc.addupdate_scatter(dout_v, [s], ones)
        plsc.addupdate_scatter(din_v, [d], ones)
        return carry

    lax.fori_loop(0, EW // 16, body, 0)
    pltpu.sync_copy(dout_v, deg_hbm.at[w, 0])
    pltpu.sync_copy(din_v, deg_hbm.at[w, 1])


@functools.partial(
    pl.kernel,
    out_type=jax.ShapeDtypeStruct((NC, NPAD, D), _f32),
    mesh=_mesh,
    compiler_params=_sc_params,
    scratch_types=[
        pltpu.VMEM((GCH, 128), jnp.int32),
        pltpu.VMEM((GCH, 128), jnp.int32),
        pltpu.VMEM((128, D), _f32),
        pltpu.VMEM((128, D), _f32),
        pltpu.VMEM_SHARED((NPAD, D), _f32),
        pltpu.SemaphoreType.DMA,
        pltpu.SemaphoreType.DMA,
    ],
)
def _msg_kernel(tab_hbm, src_hbm, dst_hbm, zeros_hbm, out_hbm,
                src_v, dst_v, rows_a, rows_b, acc, sem_a, sem_b):
    cid = lax.axis_index("c")
    sid = lax.axis_index("s")
    w = cid * NS + sid

    for k in range(ROWS_PER_TILE // 128):
        pltpu.sync_copy(zeros_hbm,
                        acc.at[pl.ds(sid * ROWS_PER_TILE + k * 128, 128)])
    plsc.subcore_barrier()

    def group(g, carry):
        pltpu.sync_copy(src_hbm.at[w, pl.ds(g * GCH, GCH)], src_v)
        pltpu.sync_copy(dst_hbm.at[w, pl.ds(g * GCH, GCH)], dst_v)

        pltpu.async_copy(tab_hbm.at[src_v.at[0]], rows_a, sem_a)

        def body(j, carry):
            even = j % 2 == 0

            @pl.when(jnp.logical_and(even, j + 1 < GCH))
            def _():
                pltpu.async_copy(tab_hbm.at[src_v.at[j + 1]], rows_b, sem_b)

            @pl.when(jnp.logical_and(jnp.logical_not(even), j + 1 < GCH))
            def _():
                pltpu.async_copy(tab_hbm.at[src_v.at[j + 1]], rows_a, sem_a)

            @pl.when(even)
            def _():
                pltpu.make_async_copy(
                    tab_hbm.at[src_v.at[j]], rows_a, sem_a).wait()
                pltpu.sync_copy(rows_a, acc.at[dst_v.at[j]], add=True)

            @pl.when(jnp.logical_not(even))
            def _():
                pltpu.make_async_copy(
                    tab_hbm.at[src_v.at[j]], rows_b, sem_b).wait()
                pltpu.sync_copy(rows_b, acc.at[dst_v.at[j]], add=True)

            return carry

        lax.fori_loop(0, GCH, body, carry)
        return carry

    lax.fori_loop(0, ECH // GCH, group, 0)
    plsc.subcore_barrier()
    pltpu.sync_copy(acc.at[pl.ds(sid * ROWS_PER_TILE, ROWS_PER_TILE)],
                    out_hbm.at[cid, pl.ds(sid * ROWS_PER_TILE, ROWS_PER_TILE)])


@functools.partial(
    pl.kernel,
    out_type=(jax.ShapeDtypeStruct((B, D), _f32),
              jax.ShapeDtypeStruct((B, D), _f32)),
    mesh=_mesh,
    compiler_params=_sc_params,
    scratch_types=[
        pltpu.VMEM((128,), jnp.int32),
        pltpu.VMEM((128,), jnp.int32),
        pltpu.VMEM((128, D), _f32),
        pltpu.VMEM((128, D), _f32),
        pltpu.SemaphoreType.DMA,
        pltpu.SemaphoreType.DMA,
    ],
)
def _gather_kernel(h2_hbm, xi_hbm, emb_hbm, ci_hbm, enc_out, emb_out,
                   xi_v, ci_v, rows_a, rows_b, sem_a, sem_b):
    cid = lax.axis_index("c")
    sid = lax.axis_index("s")
    base = (cid * NS + sid) * 128
    pltpu.sync_copy(xi_hbm.at[pl.ds(base, 128)], xi_v)
    pltpu.sync_copy(ci_hbm.at[pl.ds(base, 128)], ci_v)
    ca = pltpu.async_copy(h2_hbm.at[xi_v], rows_a, sem_a)
    cb = pltpu.async_copy(emb_hbm.at[ci_v], rows_b, sem_b)
    ca.wait()
    pltpu.sync_copy(rows_a, enc_out.at[pl.ds(base, 128)])
    cb.wait()
    pltpu.sync_copy(rows_b, emb_out.at[pl.ds(base, 128)])


def _prep_body(deg_ref, x_ref, x1_ref, rsout_ref, rsin_ref):
    deg = jnp.sum(deg_ref[...], axis=2, keepdims=True)
    rs = lax.rsqrt(jnp.maximum(deg, 1.0))
    x1_ref[...] = x_ref[...] * rs[0]
    rsout_ref[...] = jnp.broadcast_to(rs[0], (NPAD, D))
    rsin_ref[...] = jnp.broadcast_to(rs[1], (NPAD, D))


def _prep_call(deg_t, x_pad):
    return pl.pallas_call(
        _prep_body,
        out_shape=(jax.ShapeDtypeStruct((NPAD, D), _f32),
                   jax.ShapeDtypeStruct((NPAD, D), _f32),
                   jax.ShapeDtypeStruct((NPAD, D), _f32)),
    )(deg_t, x_pad)


def _dense1_body(agg_ref, rsin_ref, rsout_ref, w1_ref, b1_ref, w2_ref, g1_ref):
    a = (agg_ref[0] + agg_ref[1]) * rsin_ref[...]
    h1 = jnp.maximum(
        jnp.dot(a, w1_ref[...], preferred_element_type=_f32) + b1_ref[...],
        0.0)
    g1_ref[...] = rsout_ref[...] * jnp.dot(h1, w2_ref[...],
                                           preferred_element_type=_f32)


def _dense1_call(agg1, rsin_f, rsout_f, W1, b1_2d, W2):
    return pl.pallas_call(
        _dense1_body,
        out_shape=jax.ShapeDtypeStruct((NPAD, D), _f32),
    )(agg1, rsin_f, rsout_f, W1, b1_2d, W2)


def _dense2_body(agg_ref, rsin_ref, b2_ref, h2_ref):
    h2_ref[...] = jnp.maximum(
        (agg_ref[0] + agg_ref[1]) * rsin_ref[...] + b2_ref[...], 0.0)


def _dense2_call(agg2, rsin_f, b2_2d):
    return pl.pallas_call(
        _dense2_body,
        out_shape=jax.ShapeDtypeStruct((NPAD, D), _f32),
    )(agg2, rsin_f, b2_2d)


def _final_body(emb_ref, enc_ref, wp_ref, bp_ref, out_ref):
    p = jnp.dot(enc_ref[...], wp_ref[...], preferred_element_type=_f32)
    p = p + bp_ref[...]
    out_ref[...] = lax.dot_general(
        emb_ref[...], p, (((1,), (1,)), ((), ())),
        preferred_element_type=_f32)


def _final_call(emb, enc, Wp, bp_2d):
    blk = 1024
    return pl.pallas_call(
        _final_body,
        grid=(B // blk,),
        in_specs=[
            pl.BlockSpec((blk, D), lambda i: (i, 0)),
            pl.BlockSpec((B, D), lambda i: (0, 0)),
            pl.BlockSpec((D, D), lambda i: (0, 0)),
            pl.BlockSpec((1, D), lambda i: (0, 0)),
        ],
        out_specs=pl.BlockSpec((blk, B), lambda i: (i, 0)),
        out_shape=jax.ShapeDtypeStruct((B, B), _f32),
    )(emb, enc, Wp, bp_2d)


def kernel(x, edge_index, x_indices, c_indices, W1, b1, W2, b2, Wp, bp,
           emb_table):
    pad = jnp.full((EPAD - N_EDGES,), TRASH, jnp.int32)
    src_p = jnp.concatenate([edge_index[0], pad])
    dst_p = jnp.concatenate([edge_index[1], pad])
    src3 = src_p.reshape(NW, ECH, 128)
    dst3 = dst_p.reshape(NW, ECH, 128)
    x_pad = jnp.concatenate(
        [x, jnp.zeros((NPAD - N_NODES, D), _f32)], axis=0)
    zeros128 = jnp.zeros((128, D), _f32)

    deg_p = _deg_kernel(src_p, dst_p)
    deg_t = jnp.transpose(deg_p, (1, 2, 0))
    x1, rsout_f, rsin_f = _prep_call(deg_t, x_pad)

    agg1 = _msg_kernel(x1, src3, dst3, zeros128)
    g1 = _dense1_call(agg1, rsin_f, rsout_f, W1, b1.reshape(1, HID), W2)
    agg2 = _msg_kernel(g1, src3, dst3, zeros128)
    h2 = _dense2_call(agg2, rsin_f, b2.reshape(1, D))

    enc, emb = _gather_kernel(h2, x_indices, emb_table, c_indices)
    out = _final_call(emb, enc, Wp, bp.reshape(1, D))
    return out


# speedup vs baseline: 4.2955x; 1.0941x over previous
import functools

import jax
import jax.numpy as jnp
from jax import lax
from jax.experimental import pallas as pl
from jax.experimental.pallas import tpu as pltpu
from jax.experimental.pallas import tpu_sc as plsc

N_NODES = 10000
N_EDGES = 320000
D = 128
HID = 256
N_CELL = 100000
B = 4096

NC = 2
NS = 16
NW = NC * NS

NPAD = 10240
EPAD = 327680
TRASH = 10100
EW = EPAD // NW
GCH = 16
NCHUNK = EPAD // 128
CH_SLOW = 32
CH_FAST = (NCHUNK - NS * CH_SLOW) // NS
SLOW_CID = 0
ROWS_PER_TILE = NPAD // NS

_mesh = plsc.VectorSubcoreMesh(core_axis_name="c", subcore_axis_name="s",
                               num_cores=NC, num_subcores=NS)
_f32 = jnp.float32
_sc_params = pltpu.CompilerParams(needs_layout_passes=False)


@functools.partial(
    pl.kernel,
    out_type=jax.ShapeDtypeStruct((NW, 2, NPAD), _f32),
    mesh=_mesh,
    compiler_params=_sc_params,
    scratch_types=[
        pltpu.VMEM((EW,), jnp.int32),
        pltpu.VMEM((EW,), jnp.int32),
        pltpu.VMEM((NPAD,), _f32),
        pltpu.VMEM((NPAD,), _f32),
    ],
)
def _deg_kernel(src_hbm, dst_hbm, deg_hbm, src_v, dst_v, dout_v, din_v):
    cid = lax.axis_index("c")
    sid = lax.axis_index("s")
    w = cid * NS + sid
    pltpu.sync_copy(src_hbm.at[pl.ds(w * EW, EW)], src_v)
    pltpu.sync_copy(dst_hbm.at[pl.ds(w * EW, EW)], dst_v)

    zeros = jnp.zeros((16,), _f32)

    def zbody(i, carry):
        dout_v[pl.ds(i * 16, 16)] = zeros
        din_v[pl.ds(i * 16, 16)] = zeros
        return carry

    lax.fori_loop(0, NPAD // 16, zbody, 0)

    ones = jnp.ones((16,), _f32)

    def body(i, carry):
        s = src_v[pl.ds(i * 16, 16)]
        d = dst_v[pl.ds(i * 16, 16)]
        plsc.addupdate_scatter(dout_v, [s], ones)
        plsc.addupdate_scatter(din_v, [d], ones)
        return carry

    lax.fori_loop(0, EW // 16, body, 0)
    pltpu.sync_copy(dout_v, deg_hbm.at[w, 0])
    pltpu.sync_copy(din_v, deg_hbm.at[w, 1])


@functools.partial(
    pl.kernel,
    out_type=jax.ShapeDtypeStruct((NC, NPAD, D), _f32),
    mesh=_mesh,
    compiler_params=_sc_params,
    scratch_types=[
        pltpu.VMEM((GCH, 128), jnp.int32),
        pltpu.VMEM((GCH, 128), jnp.int32),
        pltpu.VMEM((128, D), _f32),
        pltpu.VMEM((128, D), _f32),
        pltpu.VMEM_SHARED((NPAD, D), _f32),
        pltpu.SemaphoreType.DMA,
        pltpu.SemaphoreType.DMA,
    ],
)
def _msg_kernel(tab_hbm, src_hbm, dst_hbm, zeros_hbm, out_hbm,
                src_v, dst_v, rows_a, rows_b, acc, sem_a, sem_b):
    cid = lax.axis_index("c")
    sid = lax.axis_index("s")
    slow = cid == SLOW_CID
    base_chunk = jnp.where(slow, sid * CH_SLOW, NS * CH_SLOW + sid * CH_FAST)
    ngroups = jnp.where(slow, CH_SLOW // GCH, CH_FAST // GCH)

    for k in range(ROWS_PER_TILE // 128):
        pltpu.sync_copy(zeros_hbm,
                        acc.at[pl.ds(sid * ROWS_PER_TILE + k * 128, 128)])
    plsc.subcore_barrier()

    def group(g, carry):
        pltpu.sync_copy(src_hbm.at[pl.ds(base_chunk + g * GCH, GCH)], src_v)
        pltpu.sync_copy(dst_hbm.at[pl.ds(base_chunk + g * GCH, GCH)], dst_v)

        pltpu.async_copy(tab_hbm.at[src_v.at[0]], rows_a, sem_a)

        def body(j, carry):
            even = j % 2 == 0

            @pl.when(jnp.logical_and(even, j + 1 < GCH))
            def _():
                pltpu.async_copy(tab_hbm.at[src_v.at[j + 1]], rows_b, sem_b)

            @pl.when(jnp.logical_and(jnp.logical_not(even), j + 1 < GCH))
            def _():
                pltpu.async_copy(tab_hbm.at[src_v.at[j + 1]], rows_a, sem_a)

            @pl.when(even)
            def _():
                pltpu.make_async_copy(
                    tab_hbm.at[src_v.at[j]], rows_a, sem_a).wait()
                pltpu.sync_copy(rows_a, acc.at[dst_v.at[j]], add=True)

            @pl.when(jnp.logical_not(even))
            def _():
                pltpu.make_async_copy(
                    tab_hbm.at[src_v.at[j]], rows_b, sem_b).wait()
                pltpu.sync_copy(rows_b, acc.at[dst_v.at[j]], add=True)

            return carry

        lax.fori_loop(0, GCH, body, carry)
        return carry

    lax.fori_loop(0, ngroups, group, 0)
    plsc.subcore_barrier()
    pltpu.sync_copy(acc.at[pl.ds(sid * ROWS_PER_TILE, ROWS_PER_TILE)],
                    out_hbm.at[cid, pl.ds(sid * ROWS_PER_TILE, ROWS_PER_TILE)])


@functools.partial(
    pl.kernel,
    out_type=(jax.ShapeDtypeStruct((B, D), _f32),
              jax.ShapeDtypeStruct((B, D), _f32)),
    mesh=_mesh,
    compiler_params=_sc_params,
    scratch_types=[
        pltpu.VMEM((128,), jnp.int32),
        pltpu.VMEM((128,), jnp.int32),
        pltpu.VMEM((128, D), _f32),
        pltpu.VMEM((128, D), _f32),
        pltpu.SemaphoreType.DMA,
        pltpu.SemaphoreType.DMA,
    ],
)
def _gather_kernel(h2_hbm, xi_hbm, emb_hbm, ci_hbm, enc_out, emb_out,
                   xi_v, ci_v, rows_a, rows_b, sem_a, sem_b):
    cid = lax.axis_index("c")
    sid = lax.axis_index("s")
    base = (cid * NS + sid) * 128
    pltpu.sync_copy(xi_hbm.at[pl.ds(base, 128)], xi_v)
    pltpu.sync_copy(ci_hbm.at[pl.ds(base, 128)], ci_v)
    ca = pltpu.async_copy(h2_hbm.at[xi_v], rows_a, sem_a)
    cb = pltpu.async_copy(emb_hbm.at[ci_v], rows_b, sem_b)
    ca.wait()
    pltpu.sync_copy(rows_a, enc_out.at[pl.ds(base, 128)])
    cb.wait()
    pltpu.sync_copy(rows_b, emb_out.at[pl.ds(base, 128)])


def _prep_body(deg_ref, x_ref, x1_ref, rsout_ref, rsin_ref):
    deg = jnp.sum(deg_ref[...], axis=2, keepdims=True)
    rs = lax.rsqrt(jnp.maximum(deg, 1.0))
    x1_ref[...] = x_ref[...] * rs[0]
    rsout_ref[...] = jnp.broadcast_to(rs[0], (NPAD, D))
    rsin_ref[...] = jnp.broadcast_to(rs[1], (NPAD, D))


def _prep_call(deg_t, x_pad):
    return pl.pallas_call(
        _prep_body,
        out_shape=(jax.ShapeDtypeStruct((NPAD, D), _f32),
                   jax.ShapeDtypeStruct((NPAD, D), _f32),
                   jax.ShapeDtypeStruct((NPAD, D), _f32)),
    )(deg_t, x_pad)


def _dense1_body(agg_ref, rsin_ref, rsout_ref, w1_ref, b1_ref, w2_ref, g1_ref):
    a = (agg_ref[0] + agg_ref[1]) * rsin_ref[...]
    h1 = jnp.maximum(
        jnp.dot(a, w1_ref[...], preferred_element_type=_f32) + b1_ref[...],
        0.0)
    g1_ref[...] = rsout_ref[...] * jnp.dot(h1, w2_ref[...],
                                           preferred_element_type=_f32)


def _dense1_call(agg1, rsin_f, rsout_f, W1, b1_2d, W2):
    return pl.pallas_call(
        _dense1_body,
        out_shape=jax.ShapeDtypeStruct((NPAD, D), _f32),
    )(agg1, rsin_f, rsout_f, W1, b1_2d, W2)


def _dense2_body(agg_ref, rsin_ref, b2_ref, h2_ref):
    h2_ref[...] = jnp.maximum(
        (agg_ref[0] + agg_ref[1]) * rsin_ref[...] + b2_ref[...], 0.0)


def _dense2_call(agg2, rsin_f, b2_2d):
    return pl.pallas_call(
        _dense2_body,
        out_shape=jax.ShapeDtypeStruct((NPAD, D), _f32),
    )(agg2, rsin_f, b2_2d)


def _final_body(emb_ref, enc_ref, wp_ref, bp_ref, out_ref):
    p = jnp.dot(enc_ref[...], wp_ref[...], preferred_element_type=_f32)
    p = p + bp_ref[...]
    out_ref[...] = lax.dot_general(
        emb_ref[...], p, (((1,), (1,)), ((), ())),
        preferred_element_type=_f32)


def _final_call(emb, enc, Wp, bp_2d):
    blk = 1024
    return pl.pallas_call(
        _final_body,
        grid=(B // blk,),
        in_specs=[
            pl.BlockSpec((blk, D), lambda i: (i, 0)),
            pl.BlockSpec((B, D), lambda i: (0, 0)),
            pl.BlockSpec((D, D), lambda i: (0, 0)),
            pl.BlockSpec((1, D), lambda i: (0, 0)),
        ],
        out_specs=pl.BlockSpec((blk, B), lambda i: (i, 0)),
        out_shape=jax.ShapeDtypeStruct((B, B), _f32),
    )(emb, enc, Wp, bp_2d)


def kernel(x, edge_index, x_indices, c_indices, W1, b1, W2, b2, Wp, bp,
           emb_table):
    pad = jnp.full((EPAD - N_EDGES,), TRASH, jnp.int32)
    src_p = jnp.concatenate([edge_index[0], pad])
    dst_p = jnp.concatenate([edge_index[1], pad])
    src3 = src_p.reshape(NCHUNK, 128)
    dst3 = dst_p.reshape(NCHUNK, 128)
    x_pad = jnp.concatenate(
        [x, jnp.zeros((NPAD - N_NODES, D), _f32)], axis=0)
    zeros128 = jnp.zeros((128, D), _f32)

    deg_p = _deg_kernel(src_p, dst_p)
    deg_t = jnp.transpose(deg_p, (1, 2, 0))
    x1, rsout_f, rsin_f = _prep_call(deg_t, x_pad)

    agg1 = _msg_kernel(x1, src3, dst3, zeros128)
    g1 = _dense1_call(agg1, rsin_f, rsout_f, W1, b1.reshape(1, HID), W2)
    agg2 = _msg_kernel(g1, src3, dst3, zeros128)
    h2 = _dense2_call(agg2, rsin_f, b2.reshape(1, D))

    enc, emb = _gather_kernel(h2, x_indices, emb_table, c_indices)
    out = _final_call(emb, enc, Wp, bp.reshape(1, D))
    return out


# speedup vs baseline: 4.2982x; 1.0006x over previous
import functools

import jax
import jax.numpy as jnp
from jax import lax
from jax.experimental import pallas as pl
from jax.experimental.pallas import tpu as pltpu
from jax.experimental.pallas import tpu_sc as plsc

N_NODES = 10000
N_EDGES = 320000
D = 128
HID = 256
N_CELL = 100000
B = 4096

NC = 2
NS = 16
NW = NC * NS

NPAD = 10240
EPAD = 327680
TRASH = 10100
EW = EPAD // NW
GCH = 16
NCHUNK = EPAD // 128
CH_SLOW = 32
CH_FAST = (NCHUNK - NS * CH_SLOW) // NS
SLOW_CID = 1
ROWS_PER_TILE = NPAD // NS

_mesh = plsc.VectorSubcoreMesh(core_axis_name="c", subcore_axis_name="s",
                               num_cores=NC, num_subcores=NS)
_f32 = jnp.float32
_sc_params = pltpu.CompilerParams(needs_layout_passes=False)


@functools.partial(
    pl.kernel,
    out_type=jax.ShapeDtypeStruct((NW, 2, NPAD), _f32),
    mesh=_mesh,
    compiler_params=_sc_params,
    scratch_types=[
        pltpu.VMEM((EW,), jnp.int32),
        pltpu.VMEM((EW,), jnp.int32),
        pltpu.VMEM((NPAD,), _f32),
        pltpu.VMEM((NPAD,), _f32),
    ],
)
def _deg_kernel(src_hbm, dst_hbm, deg_hbm, src_v, dst_v, dout_v, din_v):
    cid = lax.axis_index("c")
    sid = lax.axis_index("s")
    w = cid * NS + sid
    pltpu.sync_copy(src_hbm.at[pl.ds(w * EW, EW)], src_v)
    pltpu.sync_copy(dst_hbm.at[pl.ds(w * EW, EW)], dst_v)

    zeros = jnp.zeros((16,), _f32)

    def zbody(i, carry):
        dout_v[pl.ds(i * 16, 16)] = zeros
        din_v[pl.ds(i * 16, 16)] = zeros
        return carry

    lax.fori_loop(0, NPAD // 16, zbody, 0)

    ones = jnp.ones((16,), _f32)

    def body(i, carry):
        s = src_v[pl.ds(i * 16, 16)]
        d = dst_v[pl.ds(i * 16, 16)]
        plsc.addupdate_scatter(dout_v, [s], ones)
        plsc.addupdate_scatter(din_v, [d], ones)
        return carry

    lax.fori_loop(0, EW // 16, body, 0)
    pltpu.sync_copy(dout_v, deg_hbm.at[w, 0])
    pltpu.sync_copy(din_v, deg_hbm.at[w, 1])


@functools.partial(
    pl.kernel,
    out_type=jax.ShapeDtypeStruct((NC, NPAD, D), _f32),
    mesh=_mesh,
    compiler_params=_sc_params,
    scratch_types=[
        pltpu.VMEM((GCH, 128), jnp.int32),
        pltpu.VMEM((GCH, 128), jnp.int32),
        pltpu.VMEM((128, D), _f32),
        pltpu.VMEM((128, D), _f32),
        pltpu.VMEM_SHARED((NPAD, D), _f32),
        pltpu.SemaphoreType.DMA,
        pltpu.SemaphoreType.DMA,
    ],
)
def _msg_kernel(tab_hbm, src_hbm, dst_hbm, zeros_hbm, out_hbm,
                src_v, dst_v, rows_a, rows_b, acc, sem_a, sem_b):
    cid = lax.axis_index("c")
    sid = lax.axis_index("s")
    slow = cid == SLOW_CID
    base_chunk = jnp.where(slow, sid * CH_SLOW, NS * CH_SLOW + sid * CH_FAST)
    ngroups = jnp.where(slow, CH_SLOW // GCH, CH_FAST // GCH)

    for k in range(ROWS_PER_TILE // 128):
        pltpu.sync_copy(zeros_hbm,
                        acc.at[pl.ds(sid * ROWS_PER_TILE + k * 128, 128)])
    plsc.subcore_barrier()

    def group(g, carry):
        pltpu.sync_copy(src_hbm.at[pl.ds(base_chunk + g * GCH, GCH)], src_v)
        pltpu.sync_copy(dst_hbm.at[pl.ds(base_chunk + g * GCH, GCH)], dst_v)

        pltpu.async_copy(tab_hbm.at[src_v.at[0]], rows_a, sem_a)

        def body(j, carry):
            even = j % 2 == 0

            @pl.when(jnp.logical_and(even, j + 1 < GCH))
            def _():
                pltpu.async_copy(tab_hbm.at[src_v.at[j + 1]], rows_b, sem_b)

            @pl.when(jnp.logical_and(jnp.logical_not(even), j + 1 < GCH))
            def _():
                pltpu.async_copy(tab_hbm.at[src_v.at[j + 1]], rows_a, sem_a)

            @pl.when(even)
            def _():
                pltpu.make_async_copy(
                    tab_hbm.at[src_v.at[j]], rows_a, sem_a).wait()
                pltpu.sync_copy(rows_a, acc.at[dst_v.at[j]], add=True)

            @pl.when(jnp.logical_not(even))
            def _():
                pltpu.make_async_copy(
                    tab_hbm.at[src_v.at[j]], rows_b, sem_b).wait()
                pltpu.sync_copy(rows_b, acc.at[dst_v.at[j]], add=True)

            return carry

        lax.fori_loop(0, GCH, body, carry)
        return carry

    lax.fori_loop(0, ngroups, group, 0)
    plsc.subcore_barrier()
    pltpu.sync_copy(acc.at[pl.ds(sid * ROWS_PER_TILE, ROWS_PER_TILE)],
                    out_hbm.at[cid, pl.ds(sid * ROWS_PER_TILE, ROWS_PER_TILE)])


@functools.partial(
    pl.kernel,
    out_type=(jax.ShapeDtypeStruct((B, D), _f32),
              jax.ShapeDtypeStruct((B, D), _f32)),
    mesh=_mesh,
    compiler_params=_sc_params,
    scratch_types=[
        pltpu.VMEM((128,), jnp.int32),
        pltpu.VMEM((128,), jnp.int32),
        pltpu.VMEM((128, D), _f32),
        pltpu.VMEM((128, D), _f32),
        pltpu.SemaphoreType.DMA,
        pltpu.SemaphoreType.DMA,
    ],
)
def _gather_kernel(h2_hbm, xi_hbm, emb_hbm, ci_hbm, enc_out, emb_out,
                   xi_v, ci_v, rows_a, rows_b, sem_a, sem_b):
    cid = lax.axis_index("c")
    sid = lax.axis_index("s")
    base = (cid * NS + sid) * 128
    pltpu.sync_copy(xi_hbm.at[pl.ds(base, 128)], xi_v)
    pltpu.sync_copy(ci_hbm.at[pl.ds(base, 128)], ci_v)
    ca = pltpu.async_copy(h2_hbm.at[xi_v], rows_a, sem_a)
    cb = pltpu.async_copy(emb_hbm.at[ci_v], rows_b, sem_b)
    ca.wait()
    pltpu.sync_copy(rows_a, enc_out.at[pl.ds(base, 128)])
    cb.wait()
    pltpu.sync_copy(rows_b, emb_out.at[pl.ds(base, 128)])


def _prep_body(deg_ref, x_ref, x1_ref, rsout_ref, rsin_ref):
    deg = jnp.sum(deg_ref[...], axis=2, keepdims=True)
    rs = lax.rsqrt(jnp.maximum(deg, 1.0))
    x1_ref[...] = x_ref[...] * rs[0]
    rsout_ref[...] = jnp.broadcast_to(rs[0], (NPAD, D))
    rsin_ref[...] = jnp.broadcast_to(rs[1], (NPAD, D))


def _prep_call(deg_t, x_pad):
    return pl.pallas_call(
        _prep_body,
        out_shape=(jax.ShapeDtypeStruct((NPAD, D), _f32),
                   jax.ShapeDtypeStruct((NPAD, D), _f32),
                   jax.ShapeDtypeStruct((NPAD, D), _f32)),
    )(deg_t, x_pad)


def _dense1_body(agg_ref, rsin_ref, rsout_ref, w1_ref, b1_ref, w2_ref, g1_ref):
    a = (agg_ref[0] + agg_ref[1]) * rsin_ref[...]
    h1 = jnp.maximum(
        jnp.dot(a, w1_ref[...], preferred_element_type=_f32) + b1_ref[...],
        0.0)
    g1_ref[...] = rsout_ref[...] * jnp.dot(h1, w2_ref[...],
                                           preferred_element_type=_f32)


def _dense1_call(agg1, rsin_f, rsout_f, W1, b1_2d, W2):
    return pl.pallas_call(
        _dense1_body,
        out_shape=jax.ShapeDtypeStruct((NPAD, D), _f32),
    )(agg1, rsin_f, rsout_f, W1, b1_2d, W2)


def _dense2_body(agg_ref, rsin_ref, b2_ref, h2_ref):
    h2_ref[...] = jnp.maximum(
        (agg_ref[0] + agg_ref[1]) * rsin_ref[...] + b2_ref[...], 0.0)


def _dense2_call(agg2, rsin_f, b2_2d):
    return pl.pallas_call(
        _dense2_body,
        out_shape=jax.ShapeDtypeStruct((NPAD, D), _f32),
    )(agg2, rsin_f, b2_2d)


def _final_body(emb_ref, enc_ref, wp_ref, bp_ref, out_ref):
    p = jnp.dot(enc_ref[...], wp_ref[...], preferred_element_type=_f32)
    p = p + bp_ref[...]
    out_ref[...] = lax.dot_general(
        emb_ref[...], p, (((1,), (1,)), ((), ())),
        preferred_element_type=_f32)


def _final_call(emb, enc, Wp, bp_2d):
    blk = 1024
    return pl.pallas_call(
        _final_body,
        grid=(B // blk,),
        in_specs=[
            pl.BlockSpec((blk, D), lambda i: (i, 0)),
            pl.BlockSpec((B, D), lambda i: (0, 0)),
            pl.BlockSpec((D, D), lambda i: (0, 0)),
            pl.BlockSpec((1, D), lambda i: (0, 0)),
        ],
        out_specs=pl.BlockSpec((blk, B), lambda i: (i, 0)),
        out_shape=jax.ShapeDtypeStruct((B, B), _f32),
    )(emb, enc, Wp, bp_2d)


def kernel(x, edge_index, x_indices, c_indices, W1, b1, W2, b2, Wp, bp,
           emb_table):
    pad = jnp.full((EPAD - N_EDGES,), TRASH, jnp.int32)
    src_p = jnp.concatenate([edge_index[0], pad])
    dst_p = jnp.concatenate([edge_index[1], pad])
    src3 = src_p.reshape(NCHUNK, 128)
    dst3 = dst_p.reshape(NCHUNK, 128)
    x_pad = jnp.concatenate(
        [x, jnp.zeros((NPAD - N_NODES, D), _f32)], axis=0)
    zeros128 = jnp.zeros((128, D), _f32)

    deg_p = _deg_kernel(src_p, dst_p)
    deg_t = jnp.transpose(deg_p, (1, 2, 0))
    x1, rsout_f, rsin_f = _prep_call(deg_t, x_pad)

    agg1 = _msg_kernel(x1, src3, dst3, zeros128)
    g1 = _dense1_call(agg1, rsin_f, rsout_f, W1, b1.reshape(1, HID), W2)
    agg2 = _msg_kernel(g1, src3, dst3, zeros128)
    h2 = _dense2_call(agg2, rsin_f, b2.reshape(1, D))

    enc, emb = _gather_kernel(h2, x_indices, emb_table, c_indices)
    out = _final_call(emb, enc, Wp, bp.reshape(1, D))
    return out


# speedup vs baseline: 4.4253x; 1.0296x over previous
import functools

import jax
import jax.numpy as jnp
from jax import lax
from jax.experimental import pallas as pl
from jax.experimental.pallas import tpu as pltpu
from jax.experimental.pallas import tpu_sc as plsc

N_NODES = 10000
N_EDGES = 320000
D = 128
HID = 256
N_CELL = 100000
B = 4096

NC = 2
NS = 16
NW = NC * NS

NPAD = 10240
EPAD = 327680
TRASH = 10100
EW = EPAD // NW
GCH = 16
NCHUNK = EPAD // 128
CH_SLOW = 32
CH_FAST = (NCHUNK - NS * CH_SLOW) // NS
SLOW_CID = 1
ROWS_PER_TILE = NPAD // NS

_mesh = plsc.VectorSubcoreMesh(core_axis_name="c", subcore_axis_name="s",
                               num_cores=NC, num_subcores=NS)
_f32 = jnp.float32
_sc_params = pltpu.CompilerParams(needs_layout_passes=False)
_sc_params_nt = pltpu.CompilerParams(needs_layout_passes=False,
                                     use_tc_tiling_on_sc=False)


@functools.partial(
    pl.kernel,
    out_type=jax.ShapeDtypeStruct((NW, 2, NPAD), _f32),
    mesh=_mesh,
    compiler_params=_sc_params,
    scratch_types=[
        pltpu.VMEM((EW,), jnp.int32),
        pltpu.VMEM((EW,), jnp.int32),
        pltpu.VMEM((NPAD,), _f32),
        pltpu.VMEM((NPAD,), _f32),
    ],
)
def _deg_kernel(src_hbm, dst_hbm, deg_hbm, src_v, dst_v, dout_v, din_v):
    cid = lax.axis_index("c")
    sid = lax.axis_index("s")
    w = cid * NS + sid
    pltpu.sync_copy(src_hbm.at[pl.ds(w * EW, EW)], src_v)
    pltpu.sync_copy(dst_hbm.at[pl.ds(w * EW, EW)], dst_v)

    zeros = jnp.zeros((16,), _f32)

    def zbody(i, carry):
        dout_v[pl.ds(i * 16, 16)] = zeros
        din_v[pl.ds(i * 16, 16)] = zeros
        return carry

    lax.fori_loop(0, NPAD // 16, zbody, 0)

    ones = jnp.ones((16,), _f32)

    def body(i, carry):
        s = src_v[pl.ds(i * 16, 16)]
        d = dst_v[pl.ds(i * 16, 16)]
        plsc.addupdate_scatter(dout_v, [s], ones)
        plsc.addupdate_scatter(din_v, [d], ones)
        return carry

    lax.fori_loop(0, EW // 16, body, 0)
    pltpu.sync_copy(dout_v, deg_hbm.at[w, 0])
    pltpu.sync_copy(din_v, deg_hbm.at[w, 1])


@functools.partial(
    pl.kernel,
    out_type=jax.ShapeDtypeStruct((NC, NPAD, D), _f32),
    mesh=_mesh,
    compiler_params=_sc_params_nt,
    scratch_types=[
        pltpu.VMEM((GCH, 128), jnp.int32),
        pltpu.VMEM((GCH, 128), jnp.int32),
        pltpu.VMEM((128, D), jnp.bfloat16),
        pltpu.VMEM((128, D), jnp.bfloat16),
        pltpu.VMEM((128, D), _f32),
        pltpu.VMEM_SHARED((NPAD, D), _f32),
        pltpu.SemaphoreType.DMA,
        pltpu.SemaphoreType.DMA,
    ],
)
def _msg_kernel(tab_hbm, src_hbm, dst_hbm, zeros_hbm, out_hbm,
                src_v, dst_v, rb_a, rb_b, st, acc, sem_a, sem_b):
    cid = lax.axis_index("c")
    sid = lax.axis_index("s")
    slow = cid == SLOW_CID
    base_chunk = jnp.where(slow, sid * CH_SLOW, NS * CH_SLOW + sid * CH_FAST)
    ngroups = jnp.where(slow, CH_SLOW // GCH, CH_FAST // GCH)

    for k in range(ROWS_PER_TILE // 128):
        pltpu.sync_copy(zeros_hbm,
                        acc.at[pl.ds(sid * ROWS_PER_TILE + k * 128, 128)])
    plsc.subcore_barrier()

    def _convert(rb):
        def crow(r, carry):
            for r2 in range(2):
                for k in range(D // 32):
                    v = rb[2 * r + r2, pl.ds(32 * k, 32)]
                    a, b = plsc.unpack(v, format=plsc.PackFormat.INTERLEAVED)
                    st[2 * r + r2, pl.ds(32 * k, 16)] = a
                    st[2 * r + r2, pl.ds(32 * k + 16, 16)] = b
            return carry

        lax.fori_loop(0, 64, crow, 0)

    def group(g, carry):
        pltpu.sync_copy(src_hbm.at[pl.ds(base_chunk + g * GCH, GCH)], src_v)
        pltpu.sync_copy(dst_hbm.at[pl.ds(base_chunk + g * GCH, GCH)], dst_v)

        pltpu.async_copy(tab_hbm.at[src_v.at[0]], rb_a, sem_a)

        def body(j, carry):
            even = j % 2 == 0

            @pl.when(jnp.logical_and(even, j + 1 < GCH))
            def _():
                pltpu.async_copy(tab_hbm.at[src_v.at[j + 1]], rb_b, sem_b)

            @pl.when(jnp.logical_and(jnp.logical_not(even), j + 1 < GCH))
            def _():
                pltpu.async_copy(tab_hbm.at[src_v.at[j + 1]], rb_a, sem_a)

            @pl.when(even)
            def _():
                pltpu.make_async_copy(
                    tab_hbm.at[src_v.at[j]], rb_a, sem_a).wait()
                _convert(rb_a)
                pltpu.sync_copy(st, acc.at[dst_v.at[j]], add=True)

            @pl.when(jnp.logical_not(even))
            def _():
                pltpu.make_async_copy(
                    tab_hbm.at[src_v.at[j]], rb_b, sem_b).wait()
                _convert(rb_b)
                pltpu.sync_copy(st, acc.at[dst_v.at[j]], add=True)

            return carry

        lax.fori_loop(0, GCH, body, carry)
        return carry

    lax.fori_loop(0, ngroups, group, 0)
    plsc.subcore_barrier()
    pltpu.sync_copy(acc.at[pl.ds(sid * ROWS_PER_TILE, ROWS_PER_TILE)],
                    out_hbm.at[cid, pl.ds(sid * ROWS_PER_TILE, ROWS_PER_TILE)])


@functools.partial(
    pl.kernel,
    out_type=(jax.ShapeDtypeStruct((B, D), _f32),
              jax.ShapeDtypeStruct((B, D), _f32)),
    mesh=_mesh,
    compiler_params=_sc_params,
    scratch_types=[
        pltpu.VMEM((128,), jnp.int32),
        pltpu.VMEM((128,), jnp.int32),
        pltpu.VMEM((128, D), _f32),
        pltpu.VMEM((128, D), _f32),
        pltpu.SemaphoreType.DMA,
        pltpu.SemaphoreType.DMA,
    ],
)
def _gather_kernel(h2_hbm, xi_hbm, emb_hbm, ci_hbm, enc_out, emb_out,
                   xi_v, ci_v, rows_a, rows_b, sem_a, sem_b):
    cid = lax.axis_index("c")
    sid = lax.axis_index("s")
    base = (cid * NS + sid) * 128
    pltpu.sync_copy(xi_hbm.at[pl.ds(base, 128)], xi_v)
    pltpu.sync_copy(ci_hbm.at[pl.ds(base, 128)], ci_v)
    ca = pltpu.async_copy(h2_hbm.at[xi_v], rows_a, sem_a)
    cb = pltpu.async_copy(emb_hbm.at[ci_v], rows_b, sem_b)
    ca.wait()
    pltpu.sync_copy(rows_a, enc_out.at[pl.ds(base, 128)])
    cb.wait()
    pltpu.sync_copy(rows_b, emb_out.at[pl.ds(base, 128)])


def _prep_body(deg_ref, x_ref, x1_ref, rsout_ref, rsin_ref):
    deg = jnp.sum(deg_ref[...], axis=2, keepdims=True)
    rs = lax.rsqrt(jnp.maximum(deg, 1.0))
    x1_ref[...] = (x_ref[...] * rs[0]).astype(jnp.bfloat16)
    rsout_ref[...] = jnp.broadcast_to(rs[0], (NPAD, D))
    rsin_ref[...] = jnp.broadcast_to(rs[1], (NPAD, D))


def _prep_call(deg_t, x_pad):
    return pl.pallas_call(
        _prep_body,
        out_shape=(jax.ShapeDtypeStruct((NPAD, D), jnp.bfloat16),
                   jax.ShapeDtypeStruct((NPAD, D), _f32),
                   jax.ShapeDtypeStruct((NPAD, D), _f32)),
    )(deg_t, x_pad)


def _dense1_body(agg_ref, rsin_ref, rsout_ref, w1_ref, b1_ref, w2_ref, g1_ref):
    a = (agg_ref[0] + agg_ref[1]) * rsin_ref[...]
    h1 = jnp.maximum(
        jnp.dot(a, w1_ref[...], preferred_element_type=_f32) + b1_ref[...],
        0.0)
    g1 = rsout_ref[...] * jnp.dot(h1, w2_ref[...], preferred_element_type=_f32)
    g1_ref[...] = g1.astype(jnp.bfloat16)


def _dense1_call(agg1, rsin_f, rsout_f, W1, b1_2d, W2):
    return pl.pallas_call(
        _dense1_body,
        out_shape=jax.ShapeDtypeStruct((NPAD, D), jnp.bfloat16),
    )(agg1, rsin_f, rsout_f, W1, b1_2d, W2)


def _dense2_body(agg_ref, rsin_ref, b2_ref, h2_ref):
    h2_ref[...] = jnp.maximum(
        (agg_ref[0] + agg_ref[1]) * rsin_ref[...] + b2_ref[...], 0.0)


def _dense2_call(agg2, rsin_f, b2_2d):
    return pl.pallas_call(
        _dense2_body,
        out_shape=jax.ShapeDtypeStruct((NPAD, D), _f32),
    )(agg2, rsin_f, b2_2d)


def _final_body(emb_ref, enc_ref, wp_ref, bp_ref, out_ref):
    p = jnp.dot(enc_ref[...], wp_ref[...], preferred_element_type=_f32)
    p = p + bp_ref[...]
    out_ref[...] = lax.dot_general(
        emb_ref[...], p, (((1,), (1,)), ((), ())),
        preferred_element_type=_f32)


def _final_call(emb, enc, Wp, bp_2d):
    blk = 1024
    return pl.pallas_call(
        _final_body,
        grid=(B // blk,),
        in_specs=[
            pl.BlockSpec((blk, D), lambda i: (i, 0)),
            pl.BlockSpec((B, D), lambda i: (0, 0)),
            pl.BlockSpec((D, D), lambda i: (0, 0)),
            pl.BlockSpec((1, D), lambda i: (0, 0)),
        ],
        out_specs=pl.BlockSpec((blk, B), lambda i: (i, 0)),
        out_shape=jax.ShapeDtypeStruct((B, B), _f32),
    )(emb, enc, Wp, bp_2d)


def _q_map():
    import numpy as np
    qm = np.zeros((D,), dtype=np.int32)
    for k in range(D // 32):
        for j in range(16):
            qm[32 * k + j] = 32 * k + 2 * j
            qm[32 * k + 16 + j] = 32 * k + 2 * j + 1
    return qm


def kernel(x, edge_index, x_indices, c_indices, W1, b1, W2, b2, Wp, bp,
           emb_table):
    pad = jnp.full((EPAD - N_EDGES,), TRASH, jnp.int32)
    src_p = jnp.concatenate([edge_index[0], pad])
    dst_p = jnp.concatenate([edge_index[1], pad])
    src3 = src_p.reshape(NCHUNK, 128)
    dst3 = dst_p.reshape(NCHUNK, 128)
    x_pad = jnp.concatenate(
        [x, jnp.zeros((NPAD - N_NODES, D), _f32)], axis=0)
    zeros128 = jnp.zeros((128, D), _f32)

    deg_p = _deg_kernel(src_p, dst_p)
    deg_t = jnp.transpose(deg_p, (1, 2, 0))
    x1, rsout_f, rsin_f = _prep_call(deg_t, x_pad)

    qm = jnp.asarray(_q_map())
    W1q = jnp.take(W1, qm, axis=0)
    b2q = jnp.take(b2, qm)
    Wpq = jnp.take(Wp, qm, axis=0)

    agg1 = _msg_kernel(x1, src3, dst3, zeros128)
    g1 = _dense1_call(agg1, rsin_f, rsout_f, W1q, b1.reshape(1, HID), W2)
    agg2 = _msg_kernel(g1, src3, dst3, zeros128)
    h2 = _dense2_call(agg2, rsin_f, b2q.reshape(1, D))

    enc, emb = _gather_kernel(h2, x_indices, emb_table, c_indices)
    out = _final_call(emb, enc, Wpq, bp.reshape(1, D))
    return out


# speedup vs baseline: 4.8121x; 1.0874x over previous
import functools

import jax
import jax.numpy as jnp
from jax import lax
from jax.experimental import pallas as pl
from jax.experimental.pallas import tpu as pltpu
from jax.experimental.pallas import tpu_sc as plsc

N_NODES = 10000
N_EDGES = 320000
D = 128
HID = 256
N_CELL = 100000
B = 4096

NC = 2
NS = 16
NW = NC * NS

NPAD = 10240
EPAD = 327680
TRASH = 10100
EW = EPAD // NW
GCH = 16
NCHUNK = EPAD // 128
CH_SLOW = 32
CH_FAST = (NCHUNK - NS * CH_SLOW) // NS
SLOW_CID = 1
ROWS_PER_TILE = NPAD // NS

_mesh = plsc.VectorSubcoreMesh(core_axis_name="c", subcore_axis_name="s",
                               num_cores=NC, num_subcores=NS)
_f32 = jnp.float32
_sc_params = pltpu.CompilerParams(needs_layout_passes=False)
_sc_params_nt = pltpu.CompilerParams(needs_layout_passes=False,
                                     use_tc_tiling_on_sc=False)


@functools.partial(
    pl.kernel,
    out_type=jax.ShapeDtypeStruct((NW, 2, NPAD), _f32),
    mesh=_mesh,
    compiler_params=_sc_params,
    scratch_types=[
        pltpu.VMEM((EW,), jnp.int32),
        pltpu.VMEM((EW,), jnp.int32),
        pltpu.VMEM((NPAD,), _f32),
        pltpu.VMEM((NPAD,), _f32),
    ],
)
def _deg_kernel(src_hbm, dst_hbm, deg_hbm, src_v, dst_v, dout_v, din_v):
    cid = lax.axis_index("c")
    sid = lax.axis_index("s")
    w = cid * NS + sid
    pltpu.sync_copy(src_hbm.at[pl.ds(w * EW, EW)], src_v)
    pltpu.sync_copy(dst_hbm.at[pl.ds(w * EW, EW)], dst_v)

    zeros = jnp.zeros((16,), _f32)

    def zbody(i, carry):
        dout_v[pl.ds(i * 16, 16)] = zeros
        din_v[pl.ds(i * 16, 16)] = zeros
        return carry

    lax.fori_loop(0, NPAD // 16, zbody, 0)

    ones = jnp.ones((16,), _f32)

    def body(i, carry):
        s = src_v[pl.ds(i * 16, 16)]
        d = dst_v[pl.ds(i * 16, 16)]
        plsc.addupdate_scatter(dout_v, [s], ones)
        plsc.addupdate_scatter(din_v, [d], ones)
        return carry

    lax.fori_loop(0, EW // 16, body, 0)
    pltpu.sync_copy(dout_v, deg_hbm.at[w, 0])
    pltpu.sync_copy(din_v, deg_hbm.at[w, 1])


@functools.partial(
    pl.kernel,
    out_type=jax.ShapeDtypeStruct((NC, NPAD, D), _f32),
    mesh=_mesh,
    compiler_params=_sc_params_nt,
    scratch_types=[
        pltpu.VMEM((GCH, 128), jnp.int32),
        pltpu.VMEM((2 * GCH, 64), jnp.int32),
        pltpu.VMEM((128, D), jnp.bfloat16),
        pltpu.VMEM((128, D), jnp.bfloat16),
        pltpu.VMEM((64, D), _f32),
        pltpu.VMEM((64, D), _f32),
        pltpu.SemaphoreType.DMA,
        pltpu.SemaphoreType.DMA,
        pltpu.SemaphoreType.DMA,
        pltpu.SemaphoreType.DMA,
        pltpu.VMEM_SHARED((NPAD, D), _f32),
    ],
)
def _msg_kernel(tab_hbm, src_hbm, dst_hbm, zeros_hbm, out_hbm,
                src_v, dst_v, rb_a, rb_b, st_a, st_b,
                sem_a, sem_b, sem_sa, sem_sb, acc):
    cid = lax.axis_index("c")
    sid = lax.axis_index("s")
    slow = cid == SLOW_CID
    base_chunk = jnp.where(slow, sid * CH_SLOW, NS * CH_SLOW + sid * CH_FAST)
    ngroups = jnp.where(slow, CH_SLOW // GCH, CH_FAST // GCH)

    for k in range(ROWS_PER_TILE // 128):
        pltpu.sync_copy(zeros_hbm,
                        acc.at[pl.ds(sid * ROWS_PER_TILE + k * 128, 128)])
    plsc.subcore_barrier()

    def _convert(rb, half, st):
        def crow(i, carry):
            for rr in range(4):
                r = 4 * i + rr
                for k in range(D // 32):
                    v = rb[64 * half + r, pl.ds(32 * k, 32)]
                    a, b = plsc.unpack(v, format=plsc.PackFormat.INTERLEAVED)
                    st[r, pl.ds(32 * k, 16)] = a
                    st[r, pl.ds(32 * k + 16, 16)] = b
            return carry

        lax.fori_loop(0, 16, crow, 0)

    def _consume(j, rb):
        _convert(rb, 0, st_a)
        pltpu.async_copy(st_a, acc.at[dst_v.at[2 * j]], sem_sa, add=True)
        _convert(rb, 1, st_b)
        pltpu.async_copy(st_b, acc.at[dst_v.at[2 * j + 1]], sem_sb, add=True)

    def _drain(j):
        pltpu.make_async_copy(st_a, acc.at[dst_v.at[2 * j]], sem_sa).wait()
        pltpu.make_async_copy(st_b, acc.at[dst_v.at[2 * j + 1]], sem_sb).wait()

    def group(g, carry):
        pltpu.sync_copy(src_hbm.at[pl.ds(base_chunk + g * GCH, GCH)], src_v)
        pltpu.sync_copy(
            dst_hbm.at[pl.ds(2 * (base_chunk + g * GCH), 2 * GCH)], dst_v)

        pltpu.async_copy(tab_hbm.at[src_v.at[0]], rb_a, sem_a)

        def body(j, carry):
            even = j % 2 == 0

            @pl.when(jnp.logical_and(even, j + 1 < GCH))
            def _():
                pltpu.async_copy(tab_hbm.at[src_v.at[j + 1]], rb_b, sem_b)

            @pl.when(jnp.logical_and(jnp.logical_not(even), j + 1 < GCH))
            def _():
                pltpu.async_copy(tab_hbm.at[src_v.at[j + 1]], rb_a, sem_a)

            @pl.when(j > 0)
            def _():
                _drain(j - 1)

            @pl.when(even)
            def _():
                pltpu.make_async_copy(
                    tab_hbm.at[src_v.at[j]], rb_a, sem_a).wait()
                _consume(j, rb_a)

            @pl.when(jnp.logical_not(even))
            def _():
                pltpu.make_async_copy(
                    tab_hbm.at[src_v.at[j]], rb_b, sem_b).wait()
                _consume(j, rb_b)

            return carry

        lax.fori_loop(0, GCH, body, carry)
        _drain(GCH - 1)
        return carry

    lax.fori_loop(0, ngroups, group, 0)
    plsc.subcore_barrier()
    pltpu.sync_copy(acc.at[pl.ds(sid * ROWS_PER_TILE, ROWS_PER_TILE)],
                    out_hbm.at[cid, pl.ds(sid * ROWS_PER_TILE, ROWS_PER_TILE)])


@functools.partial(
    pl.kernel,
    out_type=(jax.ShapeDtypeStruct((B, D), _f32),
              jax.ShapeDtypeStruct((B, D), _f32)),
    mesh=_mesh,
    compiler_params=_sc_params,
    scratch_types=[
        pltpu.VMEM((128,), jnp.int32),
        pltpu.VMEM((128,), jnp.int32),
        pltpu.VMEM((128, D), _f32),
        pltpu.VMEM((128, D), _f32),
        pltpu.SemaphoreType.DMA,
        pltpu.SemaphoreType.DMA,
    ],
)
def _gather_kernel(h2_hbm, xi_hbm, emb_hbm, ci_hbm, enc_out, emb_out,
                   xi_v, ci_v, rows_a, rows_b, sem_a, sem_b):
    cid = lax.axis_index("c")
    sid = lax.axis_index("s")
    base = (cid * NS + sid) * 128
    pltpu.sync_copy(xi_hbm.at[pl.ds(base, 128)], xi_v)
    pltpu.sync_copy(ci_hbm.at[pl.ds(base, 128)], ci_v)
    ca = pltpu.async_copy(h2_hbm.at[xi_v], rows_a, sem_a)
    cb = pltpu.async_copy(emb_hbm.at[ci_v], rows_b, sem_b)
    ca.wait()
    pltpu.sync_copy(rows_a, enc_out.at[pl.ds(base, 128)])
    cb.wait()
    pltpu.sync_copy(rows_b, emb_out.at[pl.ds(base, 128)])


def _prep_body(deg_ref, x_ref, x1_ref, rsout_ref, rsin_ref):
    deg = jnp.sum(deg_ref[...], axis=2, keepdims=True)
    rs = lax.rsqrt(jnp.maximum(deg, 1.0))
    x1_ref[...] = (x_ref[...] * rs[0]).astype(jnp.bfloat16)
    rsout_ref[...] = jnp.broadcast_to(rs[0], (NPAD, D))
    rsin_ref[...] = jnp.broadcast_to(rs[1], (NPAD, D))


def _prep_call(deg_t, x_pad):
    return pl.pallas_call(
        _prep_body,
        out_shape=(jax.ShapeDtypeStruct((NPAD, D), jnp.bfloat16),
                   jax.ShapeDtypeStruct((NPAD, D), _f32),
                   jax.ShapeDtypeStruct((NPAD, D), _f32)),
    )(deg_t, x_pad)


def _dense1_body(agg_ref, rsin_ref, rsout_ref, w1_ref, b1_ref, w2_ref, g1_ref):
    a = (agg_ref[0] + agg_ref[1]) * rsin_ref[...]
    h1 = jnp.maximum(
        jnp.dot(a, w1_ref[...], preferred_element_type=_f32) + b1_ref[...],
        0.0)
    g1 = rsout_ref[...] * jnp.dot(h1, w2_ref[...], preferred_element_type=_f32)
    g1_ref[...] = g1.astype(jnp.bfloat16)


def _dense1_call(agg1, rsin_f, rsout_f, W1, b1_2d, W2):
    return pl.pallas_call(
        _dense1_body,
        out_shape=jax.ShapeDtypeStruct((NPAD, D), jnp.bfloat16),
    )(agg1, rsin_f, rsout_f, W1, b1_2d, W2)


def _dense2_body(agg_ref, rsin_ref, b2_ref, h2_ref):
    h2_ref[...] = jnp.maximum(
        (agg_ref[0] + agg_ref[1]) * rsin_ref[...] + b2_ref[...], 0.0)


def _dense2_call(agg2, rsin_f, b2_2d):
    return pl.pallas_call(
        _dense2_body,
        out_shape=jax.ShapeDtypeStruct((NPAD, D), _f32),
    )(agg2, rsin_f, b2_2d)


def _final_body(emb_ref, enc_ref, wp_ref, bp_ref, out_ref):
    p = jnp.dot(enc_ref[...], wp_ref[...], preferred_element_type=_f32)
    p = p + bp_ref[...]
    out_ref[...] = lax.dot_general(
        emb_ref[...], p, (((1,), (1,)), ((), ())),
        preferred_element_type=_f32)


def _final_call(emb, enc, Wp, bp_2d):
    blk = 1024
    return pl.pallas_call(
        _final_body,
        grid=(B // blk,),
        in_specs=[
            pl.BlockSpec((blk, D), lambda i: (i, 0)),
            pl.BlockSpec((B, D), lambda i: (0, 0)),
            pl.BlockSpec((D, D), lambda i: (0, 0)),
            pl.BlockSpec((1, D), lambda i: (0, 0)),
        ],
        out_specs=pl.BlockSpec((blk, B), lambda i: (i, 0)),
        out_shape=jax.ShapeDtypeStruct((B, B), _f32),
    )(emb, enc, Wp, bp_2d)


def _q_map():
    import numpy as np
    qm = np.zeros((D,), dtype=np.int32)
    for k in range(D // 32):
        for j in range(16):
            qm[32 * k + j] = 32 * k + 2 * j
            qm[32 * k + 16 + j] = 32 * k + 2 * j + 1
    return qm


def kernel(x, edge_index, x_indices, c_indices, W1, b1, W2, b2, Wp, bp,
           emb_table):
    pad = jnp.full((EPAD - N_EDGES,), TRASH, jnp.int32)
    src_p = jnp.concatenate([edge_index[0], pad])
    dst_p = jnp.concatenate([edge_index[1], pad])
    src3 = src_p.reshape(NCHUNK, 128)
    dst3 = dst_p.reshape(2 * NCHUNK, 64)
    x_pad = jnp.concatenate(
        [x, jnp.zeros((NPAD - N_NODES, D), _f32)], axis=0)
    zeros128 = jnp.zeros((128, D), _f32)

    deg_p = _deg_kernel(src_p, dst_p)
    deg_t = jnp.transpose(deg_p, (1, 2, 0))
    x1, rsout_f, rsin_f = _prep_call(deg_t, x_pad)

    qm = jnp.asarray(_q_map())
    W1q = jnp.take(W1, qm, axis=0)
    b2q = jnp.take(b2, qm)
    Wpq = jnp.take(Wp, qm, axis=0)

    agg1 = _msg_kernel(x1, src3, dst3, zeros128)
    g1 = _dense1_call(agg1, rsin_f, rsout_f, W1q, b1.reshape(1, HID), W2)
    agg2 = _msg_kernel(g1, src3, dst3, zeros128)
    h2 = _dense2_call(agg2, rsin_f, b2q.reshape(1, D))

    enc, emb = _gather_kernel(h2, x_indices, emb_table, c_indices)
    out = _final_call(emb, enc, Wpq, bp.reshape(1, D))
    return out


# speedup vs baseline: 6.2800x; 1.3050x over previous
import functools

import jax
import jax.numpy as jnp
from jax import lax
from jax.experimental import pallas as pl
from jax.experimental.pallas import tpu as pltpu
from jax.experimental.pallas import tpu_sc as plsc

N_NODES = 10000
N_EDGES = 320000
D = 128
HID = 256
N_CELL = 100000
B = 4096

NC = 2
NS = 16
NW = NC * NS

NPAD = 10240
EPAD = 327680
TRASH = 10100
EW = EPAD // NW
GCH = 16
NCHUNK = EPAD // 128
CH_SLOW = 48
CH_FAST = (NCHUNK - NS * CH_SLOW) // NS
SLOW_CID = 1
ROWS_PER_TILE = NPAD // NS

_mesh = plsc.VectorSubcoreMesh(core_axis_name="c", subcore_axis_name="s",
                               num_cores=NC, num_subcores=NS)
_f32 = jnp.float32
_sc_params = pltpu.CompilerParams(needs_layout_passes=False)
_sc_params_nt = pltpu.CompilerParams(needs_layout_passes=False,
                                     use_tc_tiling_on_sc=False)


@functools.partial(
    pl.kernel,
    out_type=jax.ShapeDtypeStruct((NW, 2, NPAD), _f32),
    mesh=_mesh,
    compiler_params=_sc_params,
    scratch_types=[
        pltpu.VMEM((EW,), jnp.int32),
        pltpu.VMEM((EW,), jnp.int32),
        pltpu.VMEM((NPAD,), _f32),
        pltpu.VMEM((NPAD,), _f32),
    ],
)
def _deg_kernel(src_hbm, dst_hbm, deg_hbm, src_v, dst_v, dout_v, din_v):
    cid = lax.axis_index("c")
    sid = lax.axis_index("s")
    w = cid * NS + sid
    pltpu.sync_copy(src_hbm.at[pl.ds(w * EW, EW)], src_v)
    pltpu.sync_copy(dst_hbm.at[pl.ds(w * EW, EW)], dst_v)

    zeros = jnp.zeros((16,), _f32)

    def zbody(i, carry):
        dout_v[pl.ds(i * 16, 16)] = zeros
        din_v[pl.ds(i * 16, 16)] = zeros
        return carry

    lax.fori_loop(0, NPAD // 16, zbody, 0)

    ones = jnp.ones((16,), _f32)

    def body(i, carry):
        s = src_v[pl.ds(i * 16, 16)]
        d = dst_v[pl.ds(i * 16, 16)]
        plsc.addupdate_scatter(dout_v, [s], ones)
        plsc.addupdate_scatter(din_v, [d], ones)
        return carry

    lax.fori_loop(0, EW // 16, body, 0)
    pltpu.sync_copy(dout_v, deg_hbm.at[w, 0])
    pltpu.sync_copy(din_v, deg_hbm.at[w, 1])


@functools.partial(
    pl.kernel,
    out_type=jax.ShapeDtypeStruct((NC, NPAD, D), _f32),
    mesh=_mesh,
    compiler_params=_sc_params_nt,
    scratch_types=[
        pltpu.VMEM((GCH, 128), jnp.int32),
        pltpu.VMEM((2 * GCH, 64), jnp.int32),
        pltpu.VMEM((128, D), jnp.bfloat16),
        pltpu.VMEM((128, D), jnp.bfloat16),
        pltpu.VMEM((64, D), _f32),
        pltpu.VMEM((64, D), _f32),
        pltpu.SemaphoreType.DMA,
        pltpu.SemaphoreType.DMA,
        pltpu.SemaphoreType.DMA,
        pltpu.SemaphoreType.DMA,
        pltpu.VMEM_SHARED((NPAD, D), _f32),
    ],
)
def _msg_kernel(tab_hbm, src_hbm, dst_hbm, zeros_hbm, out_hbm,
                src_v, dst_v, rb_a, rb_b, st_a, st_b,
                sem_a, sem_b, sem_sa, sem_sb, acc):
    cid = lax.axis_index("c")
    sid = lax.axis_index("s")
    slow = cid == SLOW_CID
    base_chunk = jnp.where(slow, sid * CH_SLOW, NS * CH_SLOW + sid * CH_FAST)
    ngroups = jnp.where(slow, CH_SLOW // GCH, CH_FAST // GCH)

    for k in range(ROWS_PER_TILE // 128):
        pltpu.sync_copy(zeros_hbm,
                        acc.at[pl.ds(sid * ROWS_PER_TILE + k * 128, 128)])
    plsc.subcore_barrier()

    def _convert(rb, half, st):
        for r in range(64):
            for k in range(D // 32):
                v = rb[64 * half + r, pl.ds(32 * k, 32)]
                a, b = plsc.unpack(v, format=plsc.PackFormat.INTERLEAVED)
                st[r, pl.ds(32 * k, 16)] = a
                st[r, pl.ds(32 * k + 16, 16)] = b

    def _consume(j, rb):
        _convert(rb, 0, st_a)
        pltpu.async_copy(st_a, acc.at[dst_v.at[2 * j]], sem_sa, add=True)
        _convert(rb, 1, st_b)
        pltpu.async_copy(st_b, acc.at[dst_v.at[2 * j + 1]], sem_sb, add=True)

    def _drain(j):
        pltpu.make_async_copy(st_a, acc.at[dst_v.at[2 * j]], sem_sa).wait()
        pltpu.make_async_copy(st_b, acc.at[dst_v.at[2 * j + 1]], sem_sb).wait()

    def group(g, carry):
        pltpu.sync_copy(src_hbm.at[pl.ds(base_chunk + g * GCH, GCH)], src_v)
        pltpu.sync_copy(
            dst_hbm.at[pl.ds(2 * (base_chunk + g * GCH), 2 * GCH)], dst_v)

        pltpu.async_copy(tab_hbm.at[src_v.at[0]], rb_a, sem_a)

        def body(j, carry):
            even = j % 2 == 0

            @pl.when(jnp.logical_and(even, j + 1 < GCH))
            def _():
                pltpu.async_copy(tab_hbm.at[src_v.at[j + 1]], rb_b, sem_b)

            @pl.when(jnp.logical_and(jnp.logical_not(even), j + 1 < GCH))
            def _():
                pltpu.async_copy(tab_hbm.at[src_v.at[j + 1]], rb_a, sem_a)

            @pl.when(j > 0)
            def _():
                _drain(j - 1)

            @pl.when(even)
            def _():
                pltpu.make_async_copy(
                    tab_hbm.at[src_v.at[j]], rb_a, sem_a).wait()
                _consume(j, rb_a)

            @pl.when(jnp.logical_not(even))
            def _():
                pltpu.make_async_copy(
                    tab_hbm.at[src_v.at[j]], rb_b, sem_b).wait()
                _consume(j, rb_b)

            return carry

        lax.fori_loop(0, GCH, body, carry)
        _drain(GCH - 1)
        return carry

    lax.fori_loop(0, ngroups, group, 0)
    plsc.subcore_barrier()
    pltpu.sync_copy(acc.at[pl.ds(sid * ROWS_PER_TILE, ROWS_PER_TILE)],
                    out_hbm.at[cid, pl.ds(sid * ROWS_PER_TILE, ROWS_PER_TILE)])


@functools.partial(
    pl.kernel,
    out_type=(jax.ShapeDtypeStruct((B, D), _f32),
              jax.ShapeDtypeStruct((B, D), _f32)),
    mesh=_mesh,
    compiler_params=_sc_params,
    scratch_types=[
        pltpu.VMEM((128,), jnp.int32),
        pltpu.VMEM((128,), jnp.int32),
        pltpu.VMEM((128, D), _f32),
        pltpu.VMEM((128, D), _f32),
        pltpu.SemaphoreType.DMA,
        pltpu.SemaphoreType.DMA,
    ],
)
def _gather_kernel(h2_hbm, xi_hbm, emb_hbm, ci_hbm, enc_out, emb_out,
                   xi_v, ci_v, rows_a, rows_b, sem_a, sem_b):
    cid = lax.axis_index("c")
    sid = lax.axis_index("s")
    base = (cid * NS + sid) * 128
    pltpu.sync_copy(xi_hbm.at[pl.ds(base, 128)], xi_v)
    pltpu.sync_copy(ci_hbm.at[pl.ds(base, 128)], ci_v)
    ca = pltpu.async_copy(h2_hbm.at[xi_v], rows_a, sem_a)
    cb = pltpu.async_copy(emb_hbm.at[ci_v], rows_b, sem_b)
    ca.wait()
    pltpu.sync_copy(rows_a, enc_out.at[pl.ds(base, 128)])
    cb.wait()
    pltpu.sync_copy(rows_b, emb_out.at[pl.ds(base, 128)])


def _prep_body(deg_ref, x_ref, x1_ref, rsout_ref, rsin_ref):
    deg = jnp.sum(deg_ref[...], axis=2, keepdims=True)
    rs = lax.rsqrt(jnp.maximum(deg, 1.0))
    x1_ref[...] = (x_ref[...] * rs[0]).astype(jnp.bfloat16)
    rsout_ref[...] = jnp.broadcast_to(rs[0], (NPAD, D))
    rsin_ref[...] = jnp.broadcast_to(rs[1], (NPAD, D))


def _prep_call(deg_t, x_pad):
    return pl.pallas_call(
        _prep_body,
        out_shape=(jax.ShapeDtypeStruct((NPAD, D), jnp.bfloat16),
                   jax.ShapeDtypeStruct((NPAD, D), _f32),
                   jax.ShapeDtypeStruct((NPAD, D), _f32)),
    )(deg_t, x_pad)


def _dense1_body(agg_ref, rsin_ref, rsout_ref, w1_ref, b1_ref, w2_ref, g1_ref):
    a = (agg_ref[0] + agg_ref[1]) * rsin_ref[...]
    h1 = jnp.maximum(
        jnp.dot(a, w1_ref[...], preferred_element_type=_f32) + b1_ref[...],
        0.0)
    g1 = rsout_ref[...] * jnp.dot(h1, w2_ref[...], preferred_element_type=_f32)
    g1_ref[...] = g1.astype(jnp.bfloat16)


def _dense1_call(agg1, rsin_f, rsout_f, W1, b1_2d, W2):
    return pl.pallas_call(
        _dense1_body,
        out_shape=jax.ShapeDtypeStruct((NPAD, D), jnp.bfloat16),
    )(agg1, rsin_f, rsout_f, W1, b1_2d, W2)


def _dense2_body(agg_ref, rsin_ref, b2_ref, h2_ref):
    h2_ref[...] = jnp.maximum(
        (agg_ref[0] + agg_ref[1]) * rsin_ref[...] + b2_ref[...], 0.0)


def _dense2_call(agg2, rsin_f, b2_2d):
    return pl.pallas_call(
        _dense2_body,
        out_shape=jax.ShapeDtypeStruct((NPAD, D), _f32),
    )(agg2, rsin_f, b2_2d)


def _final_body(emb_ref, enc_ref, wp_ref, bp_ref, out_ref):
    p = jnp.dot(enc_ref[...], wp_ref[...], preferred_element_type=_f32)
    p = p + bp_ref[...]
    out_ref[...] = lax.dot_general(
        emb_ref[...], p, (((1,), (1,)), ((), ())),
        preferred_element_type=_f32)


def _final_call(emb, enc, Wp, bp_2d):
    blk = 1024
    return pl.pallas_call(
        _final_body,
        grid=(B // blk,),
        in_specs=[
            pl.BlockSpec((blk, D), lambda i: (i, 0)),
            pl.BlockSpec((B, D), lambda i: (0, 0)),
            pl.BlockSpec((D, D), lambda i: (0, 0)),
            pl.BlockSpec((1, D), lambda i: (0, 0)),
        ],
        out_specs=pl.BlockSpec((blk, B), lambda i: (i, 0)),
        out_shape=jax.ShapeDtypeStruct((B, B), _f32),
    )(emb, enc, Wp, bp_2d)


def _q_map():
    import numpy as np
    qm = np.zeros((D,), dtype=np.int32)
    for k in range(D // 32):
        for j in range(16):
            qm[32 * k + j] = 32 * k + 2 * j
            qm[32 * k + 16 + j] = 32 * k + 2 * j + 1
    return qm


def kernel(x, edge_index, x_indices, c_indices, W1, b1, W2, b2, Wp, bp,
           emb_table):
    pad = jnp.full((EPAD - N_EDGES,), TRASH, jnp.int32)
    src_p = jnp.concatenate([edge_index[0], pad])
    dst_p = jnp.concatenate([edge_index[1], pad])
    src3 = src_p.reshape(NCHUNK, 128)
    dst3 = dst_p.reshape(2 * NCHUNK, 64)
    x_pad = jnp.concatenate(
        [x, jnp.zeros((NPAD - N_NODES, D), _f32)], axis=0)
    zeros128 = jnp.zeros((128, D), _f32)

    deg_p = _deg_kernel(src_p, dst_p)
    deg_t = jnp.transpose(deg_p, (1, 2, 0))
    x1, rsout_f, rsin_f = _prep_call(deg_t, x_pad)

    qm = jnp.asarray(_q_map())
    W1q = jnp.take(W1, qm, axis=0)
    b2q = jnp.take(b2, qm)
    Wpq = jnp.take(Wp, qm, axis=0)

    agg1 = _msg_kernel(x1, src3, dst3, zeros128)
    g1 = _dense1_call(agg1, rsin_f, rsout_f, W1q, b1.reshape(1, HID), W2)
    agg2 = _msg_kernel(g1, src3, dst3, zeros128)
    h2 = _dense2_call(agg2, rsin_f, b2q.reshape(1, D))

    enc, emb = _gather_kernel(h2, x_indices, emb_table, c_indices)
    out = _final_call(emb, enc, Wpq, bp.reshape(1, D))
    return out


# speedup vs baseline: 7.1762x; 1.1427x over previous
import functools

import jax
import jax.numpy as jnp
from jax import lax
from jax.experimental import pallas as pl
from jax.experimental.pallas import tpu as pltpu
from jax.experimental.pallas import tpu_sc as plsc

N_NODES = 10000
N_EDGES = 320000
D = 128
HID = 256
N_CELL = 100000
B = 4096

NC = 2
NS = 16
NW = NC * NS

NPAD = 10240
EPAD = 327680
TRASH = 10100
EW = EPAD // NW
GCH = 16
NCHUNK = EPAD // 128
CH_SLOW = 80
CH_FAST = (NCHUNK - NS * CH_SLOW) // NS
SLOW_CID = 1
ROWS_PER_TILE = NPAD // NS

_mesh = plsc.VectorSubcoreMesh(core_axis_name="c", subcore_axis_name="s",
                               num_cores=NC, num_subcores=NS)
_f32 = jnp.float32
_sc_params = pltpu.CompilerParams(needs_layout_passes=False)
_sc_params_nt = pltpu.CompilerParams(needs_layout_passes=False,
                                     use_tc_tiling_on_sc=False)


@functools.partial(
    pl.kernel,
    out_type=jax.ShapeDtypeStruct((NW, 2, NPAD), _f32),
    mesh=_mesh,
    compiler_params=_sc_params,
    scratch_types=[
        pltpu.VMEM((EW,), jnp.int32),
        pltpu.VMEM((EW,), jnp.int32),
        pltpu.VMEM((NPAD,), _f32),
        pltpu.VMEM((NPAD,), _f32),
    ],
)
def _deg_kernel(src_hbm, dst_hbm, deg_hbm, src_v, dst_v, dout_v, din_v):
    cid = lax.axis_index("c")
    sid = lax.axis_index("s")
    w = cid * NS + sid
    pltpu.sync_copy(src_hbm.at[pl.ds(w * EW, EW)], src_v)
    pltpu.sync_copy(dst_hbm.at[pl.ds(w * EW, EW)], dst_v)

    zeros = jnp.zeros((16,), _f32)

    def zbody(i, carry):
        dout_v[pl.ds(i * 16, 16)] = zeros
        din_v[pl.ds(i * 16, 16)] = zeros
        return carry

    lax.fori_loop(0, NPAD // 16, zbody, 0)

    ones = jnp.ones((16,), _f32)

    def body(i, carry):
        s = src_v[pl.ds(i * 16, 16)]
        d = dst_v[pl.ds(i * 16, 16)]
        plsc.addupdate_scatter(dout_v, [s], ones)
        plsc.addupdate_scatter(din_v, [d], ones)
        return carry

    lax.fori_loop(0, EW // 16, body, 0)
    pltpu.sync_copy(dout_v, deg_hbm.at[w, 0])
    pltpu.sync_copy(din_v, deg_hbm.at[w, 1])


@functools.partial(
    pl.kernel,
    out_type=jax.ShapeDtypeStruct((NC, NPAD, D), _f32),
    mesh=_mesh,
    compiler_params=_sc_params_nt,
    scratch_types=[
        pltpu.VMEM((GCH, 128), jnp.int32),
        pltpu.VMEM((2 * GCH, 64), jnp.int32),
        pltpu.VMEM((128, D), jnp.bfloat16),
        pltpu.VMEM((128, D), jnp.bfloat16),
        pltpu.VMEM((64, D), _f32),
        pltpu.VMEM((64, D), _f32),
        pltpu.SemaphoreType.DMA,
        pltpu.SemaphoreType.DMA,
        pltpu.SemaphoreType.DMA,
        pltpu.SemaphoreType.DMA,
        pltpu.VMEM_SHARED((NPAD, D), _f32),
    ],
)
def _msg_kernel(tab_hbm, src_hbm, dst_hbm, zeros_hbm, out_hbm,
                src_v, dst_v, rb_a, rb_b, st_a, st_b,
                sem_a, sem_b, sem_sa, sem_sb, acc):
    cid = lax.axis_index("c")
    sid = lax.axis_index("s")
    slow = cid == SLOW_CID
    base_chunk = jnp.where(slow, sid * CH_SLOW, NS * CH_SLOW + sid * CH_FAST)
    ngroups = jnp.where(slow, CH_SLOW // GCH, CH_FAST // GCH)

    for k in range(ROWS_PER_TILE // 128):
        pltpu.sync_copy(zeros_hbm,
                        acc.at[pl.ds(sid * ROWS_PER_TILE + k * 128, 128)])
    plsc.subcore_barrier()

    def _convert(rb, half, st):
        for r in range(64):
            for k in range(D // 32):
                v = rb[64 * half + r, pl.ds(32 * k, 32)]
                a, b = plsc.unpack(v, format=plsc.PackFormat.INTERLEAVED)
                st[r, pl.ds(32 * k, 16)] = a
                st[r, pl.ds(32 * k + 16, 16)] = b

    def _consume(j, rb):
        _convert(rb, 0, st_a)
        pltpu.async_copy(st_a, acc.at[dst_v.at[2 * j]], sem_sa, add=True)
        _convert(rb, 1, st_b)
        pltpu.async_copy(st_b, acc.at[dst_v.at[2 * j + 1]], sem_sb, add=True)

    def _drain(j):
        pltpu.make_async_copy(st_a, acc.at[dst_v.at[2 * j]], sem_sa).wait()
        pltpu.make_async_copy(st_b, acc.at[dst_v.at[2 * j + 1]], sem_sb).wait()

    def group(g, carry):
        pltpu.sync_copy(src_hbm.at[pl.ds(base_chunk + g * GCH, GCH)], src_v)
        pltpu.sync_copy(
            dst_hbm.at[pl.ds(2 * (base_chunk + g * GCH), 2 * GCH)], dst_v)

        pltpu.async_copy(tab_hbm.at[src_v.at[0]], rb_a, sem_a)

        def body(j, carry):
            even = j % 2 == 0

            @pl.when(jnp.logical_and(even, j + 1 < GCH))
            def _():
                pltpu.async_copy(tab_hbm.at[src_v.at[j + 1]], rb_b, sem_b)

            @pl.when(jnp.logical_and(jnp.logical_not(even), j + 1 < GCH))
            def _():
                pltpu.async_copy(tab_hbm.at[src_v.at[j + 1]], rb_a, sem_a)

            @pl.when(j > 0)
            def _():
                _drain(j - 1)

            @pl.when(even)
            def _():
                pltpu.make_async_copy(
                    tab_hbm.at[src_v.at[j]], rb_a, sem_a).wait()
                _consume(j, rb_a)

            @pl.when(jnp.logical_not(even))
            def _():
                pltpu.make_async_copy(
                    tab_hbm.at[src_v.at[j]], rb_b, sem_b).wait()
                _consume(j, rb_b)

            return carry

        lax.fori_loop(0, GCH, body, carry)
        _drain(GCH - 1)
        return carry

    lax.fori_loop(0, ngroups, group, 0)
    plsc.subcore_barrier()
    pltpu.sync_copy(acc.at[pl.ds(sid * ROWS_PER_TILE, ROWS_PER_TILE)],
                    out_hbm.at[cid, pl.ds(sid * ROWS_PER_TILE, ROWS_PER_TILE)])


@functools.partial(
    pl.kernel,
    out_type=(jax.ShapeDtypeStruct((B, D), _f32),
              jax.ShapeDtypeStruct((B, D), _f32)),
    mesh=_mesh,
    compiler_params=_sc_params,
    scratch_types=[
        pltpu.VMEM((128,), jnp.int32),
        pltpu.VMEM((128,), jnp.int32),
        pltpu.VMEM((128, D), _f32),
        pltpu.VMEM((128, D), _f32),
        pltpu.SemaphoreType.DMA,
        pltpu.SemaphoreType.DMA,
    ],
)
def _gather_kernel(h2_hbm, xi_hbm, emb_hbm, ci_hbm, enc_out, emb_out,
                   xi_v, ci_v, rows_a, rows_b, sem_a, sem_b):
    cid = lax.axis_index("c")
    sid = lax.axis_index("s")
    base = (cid * NS + sid) * 128
    pltpu.sync_copy(xi_hbm.at[pl.ds(base, 128)], xi_v)
    pltpu.sync_copy(ci_hbm.at[pl.ds(base, 128)], ci_v)
    ca = pltpu.async_copy(h2_hbm.at[xi_v], rows_a, sem_a)
    cb = pltpu.async_copy(emb_hbm.at[ci_v], rows_b, sem_b)
    ca.wait()
    pltpu.sync_copy(rows_a, enc_out.at[pl.ds(base, 128)])
    cb.wait()
    pltpu.sync_copy(rows_b, emb_out.at[pl.ds(base, 128)])


def _prep_body(deg_ref, x_ref, x1_ref, rsout_ref, rsin_ref):
    deg = jnp.sum(deg_ref[...], axis=2, keepdims=True)
    rs = lax.rsqrt(jnp.maximum(deg, 1.0))
    x1_ref[...] = (x_ref[...] * rs[0]).astype(jnp.bfloat16)
    rsout_ref[...] = jnp.broadcast_to(rs[0], (NPAD, D))
    rsin_ref[...] = jnp.broadcast_to(rs[1], (NPAD, D))


def _prep_call(deg_t, x_pad):
    return pl.pallas_call(
        _prep_body,
        out_shape=(jax.ShapeDtypeStruct((NPAD, D), jnp.bfloat16),
                   jax.ShapeDtypeStruct((NPAD, D), _f32),
                   jax.ShapeDtypeStruct((NPAD, D), _f32)),
    )(deg_t, x_pad)


def _dense1_body(agg_ref, rsin_ref, rsout_ref, w1_ref, b1_ref, w2_ref, g1_ref):
    a = (agg_ref[0] + agg_ref[1]) * rsin_ref[...]
    h1 = jnp.maximum(
        jnp.dot(a, w1_ref[...], preferred_element_type=_f32) + b1_ref[...],
        0.0)
    g1 = rsout_ref[...] * jnp.dot(h1, w2_ref[...], preferred_element_type=_f32)
    g1_ref[...] = g1.astype(jnp.bfloat16)


def _dense1_call(agg1, rsin_f, rsout_f, W1, b1_2d, W2):
    return pl.pallas_call(
        _dense1_body,
        out_shape=jax.ShapeDtypeStruct((NPAD, D), jnp.bfloat16),
    )(agg1, rsin_f, rsout_f, W1, b1_2d, W2)


def _dense2_body(agg_ref, rsin_ref, b2_ref, h2_ref):
    h2_ref[...] = jnp.maximum(
        (agg_ref[0] + agg_ref[1]) * rsin_ref[...] + b2_ref[...], 0.0)


def _dense2_call(agg2, rsin_f, b2_2d):
    return pl.pallas_call(
        _dense2_body,
        out_shape=jax.ShapeDtypeStruct((NPAD, D), _f32),
    )(agg2, rsin_f, b2_2d)


def _final_body(emb_ref, enc_ref, wp_ref, bp_ref, out_ref):
    p = jnp.dot(enc_ref[...], wp_ref[...], preferred_element_type=_f32)
    p = p + bp_ref[...]
    out_ref[...] = lax.dot_general(
        emb_ref[...], p, (((1,), (1,)), ((), ())),
        preferred_element_type=_f32)


def _final_call(emb, enc, Wp, bp_2d):
    blk = 1024
    return pl.pallas_call(
        _final_body,
        grid=(B // blk,),
        in_specs=[
            pl.BlockSpec((blk, D), lambda i: (i, 0)),
            pl.BlockSpec((B, D), lambda i: (0, 0)),
            pl.BlockSpec((D, D), lambda i: (0, 0)),
            pl.BlockSpec((1, D), lambda i: (0, 0)),
        ],
        out_specs=pl.BlockSpec((blk, B), lambda i: (i, 0)),
        out_shape=jax.ShapeDtypeStruct((B, B), _f32),
    )(emb, enc, Wp, bp_2d)


def _q_map():
    import numpy as np
    qm = np.zeros((D,), dtype=np.int32)
    for k in range(D // 32):
        for j in range(16):
            qm[32 * k + j] = 32 * k + 2 * j
            qm[32 * k + 16 + j] = 32 * k + 2 * j + 1
    return qm


def kernel(x, edge_index, x_indices, c_indices, W1, b1, W2, b2, Wp, bp,
           emb_table):
    pad = jnp.full((EPAD - N_EDGES,), TRASH, jnp.int32)
    src_p = jnp.concatenate([edge_index[0], pad])
    dst_p = jnp.concatenate([edge_index[1], pad])
    src3 = src_p.reshape(NCHUNK, 128)
    dst3 = dst_p.reshape(2 * NCHUNK, 64)
    x_pad = jnp.concatenate(
        [x, jnp.zeros((NPAD - N_NODES, D), _f32)], axis=0)
    zeros128 = jnp.zeros((128, D), _f32)

    deg_p = _deg_kernel(src_p, dst_p)
    deg_t = jnp.transpose(deg_p, (1, 2, 0))
    x1, rsout_f, rsin_f = _prep_call(deg_t, x_pad)

    qm = jnp.asarray(_q_map())
    W1q = jnp.take(W1, qm, axis=0)
    b2q = jnp.take(b2, qm)
    Wpq = jnp.take(Wp, qm, axis=0)

    agg1 = _msg_kernel(x1, src3, dst3, zeros128)
    g1 = _dense1_call(agg1, rsin_f, rsout_f, W1q, b1.reshape(1, HID), W2)
    agg2 = _msg_kernel(g1, src3, dst3, zeros128)
    h2 = _dense2_call(agg2, rsin_f, b2q.reshape(1, D))

    enc, emb = _gather_kernel(h2, x_indices, emb_table, c_indices)
    out = _final_call(emb, enc, Wpq, bp.reshape(1, D))
    return out


# speedup vs baseline: 7.3114x; 1.0188x over previous
import functools

import jax
import jax.numpy as jnp
from jax import lax
from jax.experimental import pallas as pl
from jax.experimental.pallas import tpu as pltpu
from jax.experimental.pallas import tpu_sc as plsc

N_NODES = 10000
N_EDGES = 320000
D = 128
HID = 256
N_CELL = 100000
B = 4096

NC = 2
NS = 16
NW = NC * NS

NPAD = 10240
EPAD = 327680
TRASH = 10100
EW = EPAD // NW
GCH = 16
NCHUNK = EPAD // 128
CH_SLOW = 80
CH_FAST = (NCHUNK - NS * CH_SLOW) // NS
SLOW_CID = 1
ROWS_PER_TILE = NPAD // NS

_mesh = plsc.VectorSubcoreMesh(core_axis_name="c", subcore_axis_name="s",
                               num_cores=NC, num_subcores=NS)
_f32 = jnp.float32
_sc_params = pltpu.CompilerParams(needs_layout_passes=False)
_sc_params_nt = pltpu.CompilerParams(needs_layout_passes=False,
                                     use_tc_tiling_on_sc=False)


@functools.partial(
    pl.kernel,
    out_type=jax.ShapeDtypeStruct((NW, 2, NPAD), _f32),
    mesh=_mesh,
    compiler_params=_sc_params,
    scratch_types=[
        pltpu.VMEM((EW,), jnp.int32),
        pltpu.VMEM((EW,), jnp.int32),
        pltpu.VMEM((NPAD,), _f32),
        pltpu.VMEM((NPAD,), _f32),
    ],
)
def _deg_kernel(src_hbm, dst_hbm, deg_hbm, src_v, dst_v, dout_v, din_v):
    cid = lax.axis_index("c")
    sid = lax.axis_index("s")
    w = cid * NS + sid
    pltpu.sync_copy(src_hbm.at[pl.ds(w * EW, EW)], src_v)
    pltpu.sync_copy(dst_hbm.at[pl.ds(w * EW, EW)], dst_v)

    zeros = jnp.zeros((16,), _f32)

    def zbody(i, carry):
        dout_v[pl.ds(i * 16, 16)] = zeros
        din_v[pl.ds(i * 16, 16)] = zeros
        return carry

    lax.fori_loop(0, NPAD // 16, zbody, 0)

    ones = jnp.ones((16,), _f32)

    def body(i, carry):
        s = src_v[pl.ds(i * 16, 16)]
        d = dst_v[pl.ds(i * 16, 16)]
        plsc.addupdate_scatter(dout_v, [s], ones)
        plsc.addupdate_scatter(din_v, [d], ones)
        return carry

    lax.fori_loop(0, EW // 16, body, 0)
    pltpu.sync_copy(dout_v, deg_hbm.at[w, 0])
    pltpu.sync_copy(din_v, deg_hbm.at[w, 1])


@functools.partial(
    pl.kernel,
    out_type=jax.ShapeDtypeStruct((NC, NPAD, D), _f32),
    mesh=_mesh,
    compiler_params=_sc_params_nt,
    scratch_types=[
        pltpu.VMEM((GCH, 128), jnp.int32),
        pltpu.VMEM((2 * GCH, 64), jnp.int32),
        pltpu.VMEM((128, D), jnp.bfloat16),
        pltpu.VMEM((128, D), jnp.bfloat16),
        pltpu.VMEM((64, D), _f32),
        pltpu.VMEM((64, D), _f32),
        pltpu.SemaphoreType.DMA,
        pltpu.SemaphoreType.DMA,
        pltpu.SemaphoreType.DMA,
        pltpu.SemaphoreType.DMA,
        pltpu.VMEM_SHARED((NPAD, D), _f32),
    ],
)
def _msg_kernel(tab_hbm, src_hbm, dst_hbm, zeros_hbm, out_hbm,
                src_v, dst_v, rb_a, rb_b, st_a, st_b,
                sem_a, sem_b, sem_sa, sem_sb, acc):
    cid = lax.axis_index("c")
    sid = lax.axis_index("s")
    slow = cid == SLOW_CID
    base_chunk = jnp.where(slow, sid * CH_SLOW, NS * CH_SLOW + sid * CH_FAST)
    ngroups = jnp.where(slow, CH_SLOW // GCH, CH_FAST // GCH)

    pltpu.sync_copy(zeros_hbm.at[pl.ds(0, 64)], st_a)
    for k in range(ROWS_PER_TILE // 64):
        pltpu.sync_copy(st_a,
                        acc.at[pl.ds(sid * ROWS_PER_TILE + k * 64, 64)])
    plsc.subcore_barrier()

    def _convert(rb, half, st):
        for r in range(64):
            for k in range(D // 32):
                v = rb[64 * half + r, pl.ds(32 * k, 32)]
                a, b = plsc.unpack(v, format=plsc.PackFormat.INTERLEAVED)
                st[r, pl.ds(32 * k, 16)] = a
                st[r, pl.ds(32 * k + 16, 16)] = b

    def _consume(j, rb):
        _convert(rb, 0, st_a)
        pltpu.async_copy(st_a, acc.at[dst_v.at[2 * j]], sem_sa, add=True)
        _convert(rb, 1, st_b)
        pltpu.async_copy(st_b, acc.at[dst_v.at[2 * j + 1]], sem_sb, add=True)

    def _drain(j):
        pltpu.make_async_copy(st_a, acc.at[dst_v.at[2 * j]], sem_sa).wait()
        pltpu.make_async_copy(st_b, acc.at[dst_v.at[2 * j + 1]], sem_sb).wait()

    def group(g, carry):
        pltpu.sync_copy(src_hbm.at[pl.ds(base_chunk + g * GCH, GCH)], src_v)
        pltpu.sync_copy(
            dst_hbm.at[pl.ds(2 * (base_chunk + g * GCH), 2 * GCH)], dst_v)

        pltpu.async_copy(tab_hbm.at[src_v.at[0]], rb_a, sem_a)

        def body(j, carry):
            even = j % 2 == 0

            @pl.when(jnp.logical_and(even, j + 1 < GCH))
            def _():
                pltpu.async_copy(tab_hbm.at[src_v.at[j + 1]], rb_b, sem_b)

            @pl.when(jnp.logical_and(jnp.logical_not(even), j + 1 < GCH))
            def _():
                pltpu.async_copy(tab_hbm.at[src_v.at[j + 1]], rb_a, sem_a)

            @pl.when(j > 0)
            def _():
                _drain(j - 1)

            @pl.when(even)
            def _():
                pltpu.make_async_copy(
                    tab_hbm.at[src_v.at[j]], rb_a, sem_a).wait()
                _consume(j, rb_a)

            @pl.when(jnp.logical_not(even))
            def _():
                pltpu.make_async_copy(
                    tab_hbm.at[src_v.at[j]], rb_b, sem_b).wait()
                _consume(j, rb_b)

            return carry

        lax.fori_loop(0, GCH, body, carry)
        _drain(GCH - 1)
        return carry

    lax.fori_loop(0, ngroups, group, 0)
    plsc.subcore_barrier()
    pltpu.sync_copy(acc.at[pl.ds(sid * ROWS_PER_TILE, ROWS_PER_TILE)],
                    out_hbm.at[cid, pl.ds(sid * ROWS_PER_TILE, ROWS_PER_TILE)])


@functools.partial(
    pl.kernel,
    out_type=(jax.ShapeDtypeStruct((B, D), _f32),
              jax.ShapeDtypeStruct((B, D), _f32)),
    mesh=_mesh,
    compiler_params=_sc_params,
    scratch_types=[
        pltpu.VMEM((128,), jnp.int32),
        pltpu.VMEM((128,), jnp.int32),
        pltpu.VMEM((128, D), _f32),
        pltpu.VMEM((128, D), _f32),
        pltpu.SemaphoreType.DMA,
        pltpu.SemaphoreType.DMA,
    ],
)
def _gather_kernel(h2_hbm, xi_hbm, emb_hbm, ci_hbm, enc_out, emb_out,
                   xi_v, ci_v, rows_a, rows_b, sem_a, sem_b):
    cid = lax.axis_index("c")
    sid = lax.axis_index("s")
    base = (cid * NS + sid) * 128
    pltpu.sync_copy(xi_hbm.at[pl.ds(base, 128)], xi_v)
    pltpu.sync_copy(ci_hbm.at[pl.ds(base, 128)], ci_v)
    ca = pltpu.async_copy(h2_hbm.at[xi_v], rows_a, sem_a)
    cb = pltpu.async_copy(emb_hbm.at[ci_v], rows_b, sem_b)
    ca.wait()
    pltpu.sync_copy(rows_a, enc_out.at[pl.ds(base, 128)])
    cb.wait()
    pltpu.sync_copy(rows_b, emb_out.at[pl.ds(base, 128)])


def _prep_body(deg_ref, x_ref, x1_ref, rsout_ref, rsin_ref):
    deg = jnp.sum(deg_ref[...], axis=2, keepdims=True)
    rs = lax.rsqrt(jnp.maximum(deg, 1.0))
    x1_ref[...] = (x_ref[...] * rs[0]).astype(jnp.bfloat16)
    rsout_ref[...] = jnp.broadcast_to(rs[0], (NPAD, D))
    rsin_ref[...] = jnp.broadcast_to(rs[1], (NPAD, D))


def _prep_call(deg_t, x_pad):
    return pl.pallas_call(
        _prep_body,
        out_shape=(jax.ShapeDtypeStruct((NPAD, D), jnp.bfloat16),
                   jax.ShapeDtypeStruct((NPAD, D), _f32),
                   jax.ShapeDtypeStruct((NPAD, D), _f32)),
    )(deg_t, x_pad)


def _dense1_body(agg_ref, rsin_ref, rsout_ref, w1_ref, b1_ref, w2_ref, g1_ref):
    a = (agg_ref[0] + agg_ref[1]) * rsin_ref[...]
    h1 = jnp.maximum(
        jnp.dot(a, w1_ref[...], preferred_element_type=_f32) + b1_ref[...],
        0.0)
    g1 = rsout_ref[...] * jnp.dot(h1, w2_ref[...], preferred_element_type=_f32)
    g1_ref[...] = g1.astype(jnp.bfloat16)


def _dense1_call(agg1, rsin_f, rsout_f, W1, b1_2d, W2):
    return pl.pallas_call(
        _dense1_body,
        out_shape=jax.ShapeDtypeStruct((NPAD, D), jnp.bfloat16),
    )(agg1, rsin_f, rsout_f, W1, b1_2d, W2)


def _dense2_body(agg_ref, rsin_ref, b2_ref, h2_ref):
    h2_ref[...] = jnp.maximum(
        (agg_ref[0] + agg_ref[1]) * rsin_ref[...] + b2_ref[...], 0.0)


def _dense2_call(agg2, rsin_f, b2_2d):
    return pl.pallas_call(
        _dense2_body,
        out_shape=jax.ShapeDtypeStruct((NPAD, D), _f32),
    )(agg2, rsin_f, b2_2d)


def _final_body(emb_ref, enc_ref, wp_ref, bp_ref, out_ref):
    p = jnp.dot(enc_ref[...], wp_ref[...], preferred_element_type=_f32)
    p = p + bp_ref[...]
    out_ref[...] = lax.dot_general(
        emb_ref[...], p, (((1,), (1,)), ((), ())),
        preferred_element_type=_f32)


def _final_call(emb, enc, Wp, bp_2d):
    blk = 1024
    return pl.pallas_call(
        _final_body,
        grid=(B // blk,),
        in_specs=[
            pl.BlockSpec((blk, D), lambda i: (i, 0)),
            pl.BlockSpec((B, D), lambda i: (0, 0)),
            pl.BlockSpec((D, D), lambda i: (0, 0)),
            pl.BlockSpec((1, D), lambda i: (0, 0)),
        ],
        out_specs=pl.BlockSpec((blk, B), lambda i: (i, 0)),
        out_shape=jax.ShapeDtypeStruct((B, B), _f32),
    )(emb, enc, Wp, bp_2d)


def _q_map():
    import numpy as np
    qm = np.zeros((D,), dtype=np.int32)
    for k in range(D // 32):
        for j in range(16):
            qm[32 * k + j] = 32 * k + 2 * j
            qm[32 * k + 16 + j] = 32 * k + 2 * j + 1
    return qm


def kernel(x, edge_index, x_indices, c_indices, W1, b1, W2, b2, Wp, bp,
           emb_table):
    pad = jnp.full((EPAD - N_EDGES,), TRASH, jnp.int32)
    src_p = jnp.concatenate([edge_index[0], pad])
    dst_p = jnp.concatenate([edge_index[1], pad])
    src3 = src_p.reshape(NCHUNK, 128)
    dst3 = dst_p.reshape(2 * NCHUNK, 64)
    x_pad = jnp.concatenate(
        [x, jnp.zeros((NPAD - N_NODES, D), _f32)], axis=0)
    zeros128 = jnp.zeros((128, D), _f32)

    deg_p = _deg_kernel(src_p, dst_p)
    deg_t = jnp.transpose(deg_p, (1, 2, 0))
    x1, rsout_f, rsin_f = _prep_call(deg_t, x_pad)

    qm = jnp.asarray(_q_map())
    W1q = jnp.take(W1, qm, axis=0)
    b2q = jnp.take(b2, qm)
    Wpq = jnp.take(Wp, qm, axis=0)

    agg1 = _msg_kernel(x1, src3, dst3, zeros128)
    g1 = _dense1_call(agg1, rsin_f, rsout_f, W1q, b1.reshape(1, HID), W2)
    agg2 = _msg_kernel(g1, src3, dst3, zeros128)
    h2 = _dense2_call(agg2, rsin_f, b2q.reshape(1, D))

    enc, emb = _gather_kernel(h2, x_indices, emb_table, c_indices)
    out = _final_call(emb, enc, Wpq, bp.reshape(1, D))
    return out


# speedup vs baseline: 7.3439x; 1.0044x over previous
import functools

import jax
import jax.numpy as jnp
from jax import lax
from jax.experimental import pallas as pl
from jax.experimental.pallas import tpu as pltpu
from jax.experimental.pallas import tpu_sc as plsc

N_NODES = 10000
N_EDGES = 320000
D = 128
HID = 256
N_CELL = 100000
B = 4096

NC = 2
NS = 16
NW = NC * NS

NPAD = 10240
EPAD = 327680
TRASH = 10100
EW_A = 12800
EW_B = 7680
GCH = 16
NCHUNK = EPAD // 128
CH_SLOW = 80
CH_FAST = (NCHUNK - NS * CH_SLOW) // NS
SLOW_CID = 1
ROWS_PER_TILE = NPAD // NS

_mesh = plsc.VectorSubcoreMesh(core_axis_name="c", subcore_axis_name="s",
                               num_cores=NC, num_subcores=NS)
_f32 = jnp.float32
_sc_params = pltpu.CompilerParams(needs_layout_passes=False)
_sc_params_nt = pltpu.CompilerParams(needs_layout_passes=False,
                                     use_tc_tiling_on_sc=False)


@functools.partial(
    pl.kernel,
    out_type=jax.ShapeDtypeStruct((NW, 2, NPAD), _f32),
    mesh=_mesh,
    compiler_params=_sc_params,
    scratch_types=[
        pltpu.VMEM((EW_A,), jnp.int32),
        pltpu.VMEM((EW_A,), jnp.int32),
        pltpu.VMEM((NPAD,), _f32),
        pltpu.VMEM((NPAD,), _f32),
    ],
)
def _deg_kernel(src_hbm, dst_hbm, deg_hbm, src_v, dst_v, dout_v, din_v):
    cid = lax.axis_index("c")
    sid = lax.axis_index("s")
    w = cid * NS + sid
    ew = jnp.where(cid == SLOW_CID, EW_B, EW_A)
    base = jnp.where(cid == SLOW_CID, NS * EW_A + sid * EW_B, sid * EW_A)

    @pl.when(cid != SLOW_CID)
    def _():
        pltpu.sync_copy(src_hbm.at[pl.ds(base, EW_A)], src_v)
        pltpu.sync_copy(dst_hbm.at[pl.ds(base, EW_A)], dst_v)

    @pl.when(cid == SLOW_CID)
    def _():
        pltpu.sync_copy(src_hbm.at[pl.ds(base, EW_B)],
                        src_v.at[pl.ds(0, EW_B)])
        pltpu.sync_copy(dst_hbm.at[pl.ds(base, EW_B)],
                        dst_v.at[pl.ds(0, EW_B)])

    zeros = jnp.zeros((16,), _f32)

    def zbody(i, carry):
        dout_v[pl.ds(i * 16, 16)] = zeros
        din_v[pl.ds(i * 16, 16)] = zeros
        return carry

    lax.fori_loop(0, NPAD // 16, zbody, 0)

    ones = jnp.ones((16,), _f32)

    def body(i, carry):
        s = src_v[pl.ds(i * 16, 16)]
        d = dst_v[pl.ds(i * 16, 16)]
        plsc.addupdate_scatter(dout_v, [s], ones)
        plsc.addupdate_scatter(din_v, [d], ones)
        return carry

    lax.fori_loop(0, ew // 16, body, 0)
    pltpu.sync_copy(dout_v, deg_hbm.at[w, 0])
    pltpu.sync_copy(din_v, deg_hbm.at[w, 1])


@functools.partial(
    pl.kernel,
    out_type=jax.ShapeDtypeStruct((NC, NPAD, D), _f32),
    mesh=_mesh,
    compiler_params=_sc_params_nt,
    scratch_types=[
        pltpu.VMEM((GCH, 128), jnp.int32),
        pltpu.VMEM((2 * GCH, 64), jnp.int32),
        pltpu.VMEM((128, D), jnp.bfloat16),
        pltpu.VMEM((128, D), jnp.bfloat16),
        pltpu.VMEM((64, D), _f32),
        pltpu.VMEM((64, D), _f32),
        pltpu.SemaphoreType.DMA,
        pltpu.SemaphoreType.DMA,
        pltpu.SemaphoreType.DMA,
        pltpu.SemaphoreType.DMA,
        pltpu.VMEM_SHARED((NPAD, D), _f32),
    ],
)
def _msg_kernel(tab_hbm, src_hbm, dst_hbm, zeros_hbm, out_hbm,
                src_v, dst_v, rb_a, rb_b, st_a, st_b,
                sem_a, sem_b, sem_sa, sem_sb, acc):
    cid = lax.axis_index("c")
    sid = lax.axis_index("s")
    slow = cid == SLOW_CID
    base_chunk = jnp.where(slow, sid * CH_SLOW, NS * CH_SLOW + sid * CH_FAST)
    ngroups = jnp.where(slow, CH_SLOW // GCH, CH_FAST // GCH)

    pltpu.sync_copy(zeros_hbm.at[pl.ds(0, 64)], st_a)
    for k in range(ROWS_PER_TILE // 64):
        pltpu.sync_copy(st_a,
                        acc.at[pl.ds(sid * ROWS_PER_TILE + k * 64, 64)])
    plsc.subcore_barrier()

    def _convert(rb, half, st):
        for r in range(64):
            for k in range(D // 32):
                v = rb[64 * half + r, pl.ds(32 * k, 32)]
                a, b = plsc.unpack(v, format=plsc.PackFormat.INTERLEAVED)
                st[r, pl.ds(32 * k, 16)] = a
                st[r, pl.ds(32 * k + 16, 16)] = b

    def _consume(j, rb):
        _convert(rb, 0, st_a)
        pltpu.async_copy(st_a, acc.at[dst_v.at[2 * j]], sem_sa, add=True)
        _convert(rb, 1, st_b)
        pltpu.async_copy(st_b, acc.at[dst_v.at[2 * j + 1]], sem_sb, add=True)

    def _drain(j):
        pltpu.make_async_copy(st_a, acc.at[dst_v.at[2 * j]], sem_sa).wait()
        pltpu.make_async_copy(st_b, acc.at[dst_v.at[2 * j + 1]], sem_sb).wait()

    def group(g, carry):
        pltpu.sync_copy(src_hbm.at[pl.ds(base_chunk + g * GCH, GCH)], src_v)
        pltpu.sync_copy(
            dst_hbm.at[pl.ds(2 * (base_chunk + g * GCH), 2 * GCH)], dst_v)

        pltpu.async_copy(tab_hbm.at[src_v.at[0]], rb_a, sem_a)

        def body(j, carry):
            even = j % 2 == 0

            @pl.when(jnp.logical_and(even, j + 1 < GCH))
            def _():
                pltpu.async_copy(tab_hbm.at[src_v.at[j + 1]], rb_b, sem_b)

            @pl.when(jnp.logical_and(jnp.logical_not(even), j + 1 < GCH))
            def _():
                pltpu.async_copy(tab_hbm.at[src_v.at[j + 1]], rb_a, sem_a)

            @pl.when(j > 0)
            def _():
                _drain(j - 1)

            @pl.when(even)
            def _():
                pltpu.make_async_copy(
                    tab_hbm.at[src_v.at[j]], rb_a, sem_a).wait()
                _consume(j, rb_a)

            @pl.when(jnp.logical_not(even))
            def _():
                pltpu.make_async_copy(
                    tab_hbm.at[src_v.at[j]], rb_b, sem_b).wait()
                _consume(j, rb_b)

            return carry

        lax.fori_loop(0, GCH, body, carry)
        _drain(GCH - 1)
        return carry

    lax.fori_loop(0, ngroups, group, 0)
    plsc.subcore_barrier()
    pltpu.sync_copy(acc.at[pl.ds(sid * ROWS_PER_TILE, ROWS_PER_TILE)],
                    out_hbm.at[cid, pl.ds(sid * ROWS_PER_TILE, ROWS_PER_TILE)])


@functools.partial(
    pl.kernel,
    out_type=(jax.ShapeDtypeStruct((B, D), _f32),
              jax.ShapeDtypeStruct((B, D), _f32)),
    mesh=_mesh,
    compiler_params=_sc_params,
    scratch_types=[
        pltpu.VMEM((128,), jnp.int32),
        pltpu.VMEM((128,), jnp.int32),
        pltpu.VMEM((128, D), _f32),
        pltpu.VMEM((128, D), _f32),
        pltpu.SemaphoreType.DMA,
        pltpu.SemaphoreType.DMA,
    ],
)
def _gather_kernel(h2_hbm, xi_hbm, emb_hbm, ci_hbm, enc_out, emb_out,
                   xi_v, ci_v, rows_a, rows_b, sem_a, sem_b):
    cid = lax.axis_index("c")
    sid = lax.axis_index("s")
    base = (cid * NS + sid) * 128
    pltpu.sync_copy(xi_hbm.at[pl.ds(base, 128)], xi_v)
    pltpu.sync_copy(ci_hbm.at[pl.ds(base, 128)], ci_v)
    ca = pltpu.async_copy(h2_hbm.at[xi_v], rows_a, sem_a)
    cb = pltpu.async_copy(emb_hbm.at[ci_v], rows_b, sem_b)
    ca.wait()
    pltpu.sync_copy(rows_a, enc_out.at[pl.ds(base, 128)])
    cb.wait()
    pltpu.sync_copy(rows_b, emb_out.at[pl.ds(base, 128)])


def _prep_body(deg_ref, x_ref, x1_ref, rsout_ref, rsin_ref):
    deg = jnp.sum(deg_ref[...], axis=2, keepdims=True)
    rs = lax.rsqrt(jnp.maximum(deg, 1.0))
    x1_ref[...] = (x_ref[...] * rs[0]).astype(jnp.bfloat16)
    rsout_ref[...] = jnp.broadcast_to(rs[0], (NPAD, D))
    rsin_ref[...] = jnp.broadcast_to(rs[1], (NPAD, D))


def _prep_call(deg_t, x_pad):
    return pl.pallas_call(
        _prep_body,
        out_shape=(jax.ShapeDtypeStruct((NPAD, D), jnp.bfloat16),
                   jax.ShapeDtypeStruct((NPAD, D), _f32),
                   jax.ShapeDtypeStruct((NPAD, D), _f32)),
    )(deg_t, x_pad)


def _dense1_body(agg_ref, rsin_ref, rsout_ref, w1_ref, b1_ref, w2_ref, g1_ref):
    a = (agg_ref[0] + agg_ref[1]) * rsin_ref[...]
    h1 = jnp.maximum(
        jnp.dot(a, w1_ref[...], preferred_element_type=_f32) + b1_ref[...],
        0.0)
    g1 = rsout_ref[...] * jnp.dot(h1, w2_ref[...], preferred_element_type=_f32)
    g1_ref[...] = g1.astype(jnp.bfloat16)


def _dense1_call(agg1, rsin_f, rsout_f, W1, b1_2d, W2):
    return pl.pallas_call(
        _dense1_body,
        out_shape=jax.ShapeDtypeStruct((NPAD, D), jnp.bfloat16),
    )(agg1, rsin_f, rsout_f, W1, b1_2d, W2)


def _dense2_body(agg_ref, rsin_ref, b2_ref, h2_ref):
    h2_ref[...] = jnp.maximum(
        (agg_ref[0] + agg_ref[1]) * rsin_ref[...] + b2_ref[...], 0.0)


def _dense2_call(agg2, rsin_f, b2_2d):
    return pl.pallas_call(
        _dense2_body,
        out_shape=jax.ShapeDtypeStruct((NPAD, D), _f32),
    )(agg2, rsin_f, b2_2d)


def _final_body(emb_ref, enc_ref, wp_ref, bp_ref, out_ref):
    p = jnp.dot(enc_ref[...], wp_ref[...], preferred_element_type=_f32)
    p = p + bp_ref[...]
    out_ref[...] = lax.dot_general(
        emb_ref[...], p, (((1,), (1,)), ((), ())),
        preferred_element_type=_f32)


def _final_call(emb, enc, Wp, bp_2d):
    blk = 1024
    return pl.pallas_call(
        _final_body,
        grid=(B // blk,),
        in_specs=[
            pl.BlockSpec((blk, D), lambda i: (i, 0)),
            pl.BlockSpec((B, D), lambda i: (0, 0)),
            pl.BlockSpec((D, D), lambda i: (0, 0)),
            pl.BlockSpec((1, D), lambda i: (0, 0)),
        ],
        out_specs=pl.BlockSpec((blk, B), lambda i: (i, 0)),
        out_shape=jax.ShapeDtypeStruct((B, B), _f32),
    )(emb, enc, Wp, bp_2d)


def _q_map():
    import numpy as np
    qm = np.zeros((D,), dtype=np.int32)
    for k in range(D // 32):
        for j in range(16):
            qm[32 * k + j] = 32 * k + 2 * j
            qm[32 * k + 16 + j] = 32 * k + 2 * j + 1
    return qm


def kernel(x, edge_index, x_indices, c_indices, W1, b1, W2, b2, Wp, bp,
           emb_table):
    pad = jnp.full((EPAD - N_EDGES,), TRASH, jnp.int32)
    src_p = jnp.concatenate([edge_index[0], pad])
    dst_p = jnp.concatenate([edge_index[1], pad])
    src3 = src_p.reshape(NCHUNK, 128)
    dst3 = dst_p.reshape(2 * NCHUNK, 64)
    x_pad = jnp.concatenate(
        [x, jnp.zeros((NPAD - N_NODES, D), _f32)], axis=0)
    zeros128 = jnp.zeros((128, D), _f32)

    deg_p = _deg_kernel(src_p, dst_p)
    deg_t = jnp.transpose(deg_p, (1, 2, 0))
    x1, rsout_f, rsin_f = _prep_call(deg_t, x_pad)

    qm = jnp.asarray(_q_map())
    W1q = jnp.take(W1, qm, axis=0)
    b2q = jnp.take(b2, qm)
    Wpq = jnp.take(Wp, qm, axis=0)

    agg1 = _msg_kernel(x1, src3, dst3, zeros128)
    g1 = _dense1_call(agg1, rsin_f, rsout_f, W1q, b1.reshape(1, HID), W2)
    agg2 = _msg_kernel(g1, src3, dst3, zeros128)
    h2 = _dense2_call(agg2, rsin_f, b2q.reshape(1, D))

    enc, emb = _gather_kernel(h2, x_indices, emb_table, c_indices)
    out = _final_call(emb, enc, Wpq, bp.reshape(1, D))
    return out
